# Initial kernel scaffold; baseline (speedup 1.0000x reference)
import jax
import jax.numpy as jnp
from jax.experimental import pallas as pl


def kernel(x, edge_index, edge_attr, batch, emb, e1w0, e1b0, e2w0, e2b0, n1w0, n1b0, n2w0, n2b0, bng0, bnb0, e1w1, e1b1, e2w1, e2b1, n1w1, n1b1, n2w1, n2b1, bng1, bnb1, e1w2, e1b2, e2w2, e2b2, n1w2, n1b2, n2w2, n2b2, bng2, bnb2, ow1, ob1, ow2, ob2):
    raise NotImplementedError("write your pallas kernel here")


# speedup vs baseline: 1.5894x; 1.5894x over previous
import functools

import jax
import jax.numpy as jnp
from jax import lax
from jax.experimental import pallas as pl
from jax.experimental.pallas import tpu as pltpu
from jax.experimental.pallas import tpu_sc as plsc

N = 50000
E = 800000
D = 64
ED = 16
H = 64
G = 256

NB = 1000
N_STEPS = N // NB
EB = 4000
E_STEPS = E // EB

NW = 32
GCH = 128
NCHUNK = E // GCH
GSLAB = (NCHUNK // NW + 1) * GCH

PAIRS = N // 4
PAD_PAIRS = 12544
ZR = PAD_PAIRS // 16


def _softplus(x):
    return jnp.maximum(x, 0.0) + jnp.log(1.0 + jnp.exp(-jnp.abs(x)))


def _pq(h_eff, wi_ref, wj_ref, e1b_ref):
    wsym = (wi_ref[...] + wj_ref[...]) * 0.5
    wasym = (wi_ref[...] - wj_ref[...]) * 0.5
    p = jnp.dot(h_eff, wsym, preferred_element_type=jnp.float32, precision=lax.Precision.HIGHEST) + e1b_ref[...] * 0.5
    q = jnp.dot(h_eff, wasym, preferred_element_type=jnp.float32, precision=lax.Precision.HIGHEST)
    t1 = jnp.concatenate([p, q], axis=1)
    t2 = jnp.concatenate([p, -q], axis=1)
    return t1, t2


def _k0_body(x_ref, emb_ref, wi_ref, wj_ref, wh_ref, e1b_ref, n1b_ref,
             h_ref, t1_ref, t2_ref, hh_ref):
    xb = x_ref[...]
    oh = (xb == lax.broadcasted_iota(jnp.int32, (NB, 100), 1)).astype(jnp.float32)
    h = jnp.dot(oh, emb_ref[...], preferred_element_type=jnp.float32, precision=lax.Precision.HIGHEST)
    h_ref[...] = h
    t1_ref[...], t2_ref[...] = _pq(h, wi_ref, wj_ref, e1b_ref)
    hh_ref[...] = jnp.dot(h, wh_ref[...], preferred_element_type=jnp.float32, precision=lax.Precision.HIGHEST) + n1b_ref[...]


def _k0(x2, emb, wi, wj, wh, e1b, n1b):
    f32 = jnp.float32
    return pl.pallas_call(
        _k0_body,
        grid=(N_STEPS,),
        in_specs=[
            pl.BlockSpec((NB, 1), lambda i: (i, 0)),
            pl.BlockSpec((100, D), lambda i: (0, 0)),
            pl.BlockSpec((D, H), lambda i: (0, 0)),
            pl.BlockSpec((D, H), lambda i: (0, 0)),
            pl.BlockSpec((D, H), lambda i: (0, 0)),
            pl.BlockSpec((1, H), lambda i: (0, 0)),
            pl.BlockSpec((1, H), lambda i: (0, 0)),
        ],
        out_specs=[
            pl.BlockSpec((NB, D), lambda i: (i, 0)),
            pl.BlockSpec((NB, 2 * H), lambda i: (i, 0)),
            pl.BlockSpec((NB, 2 * H), lambda i: (i, 0)),
            pl.BlockSpec((NB, H), lambda i: (i, 0)),
        ],
        out_shape=[
            jax.ShapeDtypeStruct((N, D), f32),
            jax.ShapeDtypeStruct((N, 2 * H), f32),
            jax.ShapeDtypeStruct((N, 2 * H), f32),
            jax.ShapeDtypeStruct((N, H), f32),
        ],
    )(x2, emb, wi, wj, wh, e1b, n1b)


def _k5_body(h_ref, st_ref, bng_ref, bnb_ref, wi_ref, wj_ref, wh_ref,
             e1b_ref, n1b_ref, t1_ref, t2_ref, hh_ref, ab_ref):
    s = st_ref[0:1, :]
    sq = st_ref[1:2, :]
    mean = s * (1.0 / N)
    var = sq * (1.0 / N) - mean * mean
    alpha = bng_ref[...] * lax.rsqrt(var + 1e-5)
    beta = bnb_ref[...] - mean * alpha
    h_eff = h_ref[...] * alpha + beta
    t1_ref[...], t2_ref[...] = _pq(h_eff, wi_ref, wj_ref, e1b_ref)
    hh_ref[...] = jnp.dot(h_eff, wh_ref[...], preferred_element_type=jnp.float32, precision=lax.Precision.HIGHEST) + n1b_ref[...]
    ab_ref[...] = jnp.concatenate([alpha, beta, jnp.zeros((6, H), jnp.float32)], axis=0)


def _k5(h, stats, bng, bnb, wi, wj, wh, e1b, n1b):
    f32 = jnp.float32
    return pl.pallas_call(
        _k5_body,
        grid=(N_STEPS,),
        in_specs=[
            pl.BlockSpec((NB, D), lambda i: (i, 0)),
            pl.BlockSpec((8, H), lambda i: (0, 0)),
            pl.BlockSpec((1, H), lambda i: (0, 0)),
            pl.BlockSpec((1, H), lambda i: (0, 0)),
            pl.BlockSpec((D, H), lambda i: (0, 0)),
            pl.BlockSpec((D, H), lambda i: (0, 0)),
            pl.BlockSpec((D, H), lambda i: (0, 0)),
            pl.BlockSpec((1, H), lambda i: (0, 0)),
            pl.BlockSpec((1, H), lambda i: (0, 0)),
        ],
        out_specs=[
            pl.BlockSpec((NB, 2 * H), lambda i: (i, 0)),
            pl.BlockSpec((NB, 2 * H), lambda i: (i, 0)),
            pl.BlockSpec((NB, H), lambda i: (i, 0)),
            pl.BlockSpec((8, H), lambda i: (0, 0)),
        ],
        out_shape=[
            jax.ShapeDtypeStruct((N, 2 * H), f32),
            jax.ShapeDtypeStruct((N, 2 * H), f32),
            jax.ShapeDtypeStruct((N, H), f32),
            jax.ShapeDtypeStruct((8, H), f32),
        ],
    )(h, stats, bng, bnb, wi, wj, wh, e1b, n1b)


def _gather_body(t1_hbm, t2_hbm, dst_hbm, src_hbm, u_hbm, dslab, sslab, rows):
    c = lax.axis_index("c")
    s = lax.axis_index("s")
    wid = s * 2 + c
    cb = wid * NCHUNK // NW
    ce = (wid + 1) * NCHUNK // NW
    nch = ce - cb
    base = pl.multiple_of(cb * GCH, GCH)
    pltpu.sync_copy(dst_hbm.at[pl.ds(base, GSLAB - GCH)], dslab.at[pl.ds(0, GSLAB - GCH)])
    pltpu.sync_copy(src_hbm.at[pl.ds(base, GSLAB - GCH)], sslab.at[pl.ds(0, GSLAB - GCH)])

    @pl.when(nch * GCH == GSLAB)
    def _():
        off = pl.multiple_of(base + GSLAB - GCH, GCH)
        pltpu.sync_copy(dst_hbm.at[pl.ds(off, GCH)], dslab.at[pl.ds(GSLAB - GCH, GCH)])
        pltpu.sync_copy(src_hbm.at[pl.ds(off, GCH)], sslab.at[pl.ds(GSLAB - GCH, GCH)])

    def chunk(j, carry):
        off = pl.multiple_of(j * GCH, GCH)
        pltpu.sync_copy(t1_hbm.at[dslab.at[pl.ds(off, GCH)]], rows)
        pltpu.sync_copy(t2_hbm.at[sslab.at[pl.ds(off, GCH)]], rows, add=True)
        wb = pl.multiple_of(base + off, GCH)
        pltpu.sync_copy(rows, u_hbm.at[pl.ds(wb, GCH)])
        return carry

    lax.fori_loop(0, nch, chunk, 0)


def _k1(t1, t2, dst, src):
    mesh = plsc.VectorSubcoreMesh(core_axis_name="c", subcore_axis_name="s")
    fn = functools.partial(
        pl.kernel,
        out_type=jax.ShapeDtypeStruct((E, 2 * H), jnp.float32),
        mesh=mesh,
        scratch_types=[
            pltpu.VMEM((GSLAB,), jnp.int32),
            pltpu.VMEM((GSLAB,), jnp.int32),
            pltpu.VMEM((GCH, 2 * H), jnp.float32),
        ],
    )(_gather_body)
    return fn(t1, t2, dst, src)


def _k2_body(u_ref, ea_ref, d_ref, we_ref, e2w_ref, e2b_ref, ms_ref):
    u = u_ref[...]
    t = (u[:, 0:H] + u[:, H:2 * H]
         + jnp.dot(ea_ref[...], we_ref[...], preferred_element_type=jnp.float32, precision=lax.Precision.HIGHEST))
    m = _softplus(t)
    m2 = _softplus(
        jnp.dot(m, e2w_ref[...], preferred_element_type=jnp.float32, precision=lax.Precision.HIGHEST) + e2b_ref[...])
    even = (d_ref[...] % 2) == 0
    z = jnp.zeros((EB, H), jnp.float32)
    ms_ref[...] = jnp.concatenate(
        [jnp.where(even, m2, z), jnp.where(even, z, m2)], axis=1)


def _k2(u, ea, dst2, we, e2w, e2b):
    f32 = jnp.float32
    return pl.pallas_call(
        _k2_body,
        grid=(E_STEPS,),
        in_specs=[
            pl.BlockSpec((EB, 2 * H), lambda i: (i, 0)),
            pl.BlockSpec((EB, ED), lambda i: (i, 0)),
            pl.BlockSpec((EB, 1), lambda i: (i, 0)),
            pl.BlockSpec((ED, H), lambda i: (0, 0)),
            pl.BlockSpec((H, H), lambda i: (0, 0)),
            pl.BlockSpec((1, H), lambda i: (0, 0)),
        ],
        out_specs=pl.BlockSpec((EB, 2 * H), lambda i: (i, 0)),
        out_shape=jax.ShapeDtypeStruct((E, 2 * H), f32),
    )(u, ea, dst2, we, e2w, e2b)


def _scatter_body(ms_hbm, dst_hbm, out0_hbm, out1_hbm,
                  aggr_sh, idxb, lidx, datab):
    c = lax.axis_index("c")
    s = lax.axis_index("s")

    def zr(r, carry):
        for g in range(8):
            datab[r, pl.ds(g * 16, 16)] = jnp.zeros((16,), jnp.float32)
        return carry
    lax.fori_loop(0, GCH, zr, 0)

    zbase = pl.multiple_of(s * ZR, 8)

    def zc(k, carry):
        pltpu.sync_copy(datab, aggr_sh.at[pl.ds(pl.multiple_of(zbase + k * GCH, 8), GCH)])
        return carry
    lax.fori_loop(0, ZR // GCH, zc, 0)
    zrem = ZR - (ZR // GCH) * GCH
    pltpu.sync_copy(datab.at[pl.ds(0, zrem)],
                    aggr_sh.at[pl.ds(pl.multiple_of(zbase + (ZR // GCH) * GCH, 8), zrem)])
    plsc.subcore_barrier()

    cb = s * NCHUNK // 16
    ce = (s + 1) * NCHUNK // 16
    lo = c * PAIRS

    def remap():
        for g in range(8):
            v = idxb[pl.ds(g * 16, 16)]
            local = lax.shift_right_logical(v, 1) - lo
            ok = (local >= 0) & (local < PAIRS)
            trash = PAIRS + (v & 7)
            lidx[pl.ds(g * 16, 16)] = jnp.where(ok, local, trash)

    def chunk(j, carry):
        eoff = pl.multiple_of(j * GCH, GCH)
        pltpu.sync_copy(dst_hbm.at[pl.ds(eoff, GCH)], idxb)
        remap()
        pltpu.sync_copy(ms_hbm.at[pl.ds(eoff, GCH)], datab)
        pltpu.sync_copy(datab, aggr_sh.at[lidx], add=True)
        return carry

    lax.fori_loop(cb, ce, chunk, 0)
    plsc.subcore_barrier()

    obase = pl.multiple_of(s * ZR, 8)

    @pl.when(c == 0)
    def _():
        pltpu.sync_copy(aggr_sh.at[pl.ds(obase, ZR)], out0_hbm.at[pl.ds(obase, ZR)])

    @pl.when(c == 1)
    def _():
        pltpu.sync_copy(aggr_sh.at[pl.ds(obase, ZR)], out1_hbm.at[pl.ds(obase, ZR)])


def _k3(ms, dst):
    f32 = jnp.float32
    mesh = plsc.VectorSubcoreMesh(core_axis_name="c", subcore_axis_name="s")
    fn = functools.partial(
        pl.kernel,
        out_type=[
            jax.ShapeDtypeStruct((PAD_PAIRS, 2 * H), f32),
            jax.ShapeDtypeStruct((PAD_PAIRS, 2 * H), f32),
        ],
        mesh=mesh,
        scratch_types=[
            pltpu.VMEM_SHARED((PAD_PAIRS, 2 * H), f32),
            pltpu.VMEM((GCH,), jnp.int32),
            pltpu.VMEM((GCH,), jnp.int32),
            pltpu.VMEM((GCH, 2 * H), f32),
        ],
    )(_scatter_body)
    return fn(ms, dst)


def _k4_body(h_ref, ab_ref, hh_ref, ag_ref, wa_ref, n2w_ref, n2b_ref,
             hn_ref, st_ref):
    i = pl.program_id(0)
    alpha = ab_ref[0:1, :]
    beta = ab_ref[1:2, :]
    h_eff = h_ref[...] * alpha + beta
    t = _softplus(hh_ref[...]
                  + jnp.dot(ag_ref[...], wa_ref[...], preferred_element_type=jnp.float32, precision=lax.Precision.HIGHEST))
    u = jnp.dot(t, n2w_ref[...], preferred_element_type=jnp.float32, precision=lax.Precision.HIGHEST) + n2b_ref[...]
    hn = u + h_eff
    hn_ref[...] = hn
    upd = jnp.concatenate([
        jnp.sum(hn, axis=0, keepdims=True),
        jnp.sum(hn * hn, axis=0, keepdims=True),
        jnp.zeros((6, H), jnp.float32),
    ], axis=0)

    @pl.when(i == 0)
    def _():
        st_ref[...] = upd

    @pl.when(i > 0)
    def _():
        st_ref[...] += upd


def _k4(h, ab, hh, aggr, wa, n2w, n2b):
    f32 = jnp.float32
    return pl.pallas_call(
        _k4_body,
        grid=(N_STEPS,),
        in_specs=[
            pl.BlockSpec((NB, D), lambda i: (i, 0)),
            pl.BlockSpec((8, H), lambda i: (0, 0)),
            pl.BlockSpec((NB, H), lambda i: (i, 0)),
            pl.BlockSpec((NB, H), lambda i: (i, 0)),
            pl.BlockSpec((H, H), lambda i: (0, 0)),
            pl.BlockSpec((H, D), lambda i: (0, 0)),
            pl.BlockSpec((1, D), lambda i: (0, 0)),
        ],
        out_specs=[
            pl.BlockSpec((NB, D), lambda i: (i, 0)),
            pl.BlockSpec((8, D), lambda i: (0, 0)),
        ],
        out_shape=[
            jax.ShapeDtypeStruct((N, D), f32),
            jax.ShapeDtypeStruct((8, D), f32),
        ],
    )(h, ab, hh, aggr, wa, n2w, n2b)


def _k6_body(h_ref, st_ref, bng_ref, bnb_ref, b_ref, ow1_ref, ob1_ref,
             ow2_ref, ob2_ref, out_ref, sums, counts):
    i = pl.program_id(0)
    s = st_ref[0:1, :]
    sq = st_ref[1:2, :]
    mean = s * (1.0 / N)
    var = sq * (1.0 / N) - mean * mean
    alpha = bng_ref[...] * lax.rsqrt(var + 1e-5)
    beta = bnb_ref[...] - mean * alpha
    h_eff = h_ref[...] * alpha + beta
    bb = b_ref[...]
    oh = (bb == lax.broadcasted_iota(jnp.int32, (NB, G), 1)).astype(jnp.float32)
    dnums = (((0,), (0,)), ((), ()))
    sum_c = lax.dot_general(oh, h_eff, dnums, preferred_element_type=jnp.float32, precision=lax.Precision.HIGHEST)
    cnt_c = lax.dot_general(oh, jnp.ones((NB, 1), jnp.float32), dnums,
                            preferred_element_type=jnp.float32, precision=lax.Precision.HIGHEST)

    @pl.when(i == 0)
    def _():
        sums[...] = sum_c
        counts[...] = cnt_c

    @pl.when(i > 0)
    def _():
        sums[...] += sum_c
        counts[...] += cnt_c

    @pl.when(i == N_STEPS - 1)
    def _():
        pooled = sums[...] / jnp.maximum(counts[...], 1.0)
        o1 = _softplus(jnp.dot(pooled, ow1_ref[...], preferred_element_type=jnp.float32, precision=lax.Precision.HIGHEST)
                       + ob1_ref[...])
        out_ref[...] = jnp.dot(o1, ow2_ref[...], preferred_element_type=jnp.float32, precision=lax.Precision.HIGHEST) + ob2_ref[...]


def _k6(h, stats, bng, bnb, batch2, ow1, ob1, ow2, ob2):
    f32 = jnp.float32
    return pl.pallas_call(
        _k6_body,
        grid=(N_STEPS,),
        in_specs=[
            pl.BlockSpec((NB, D), lambda i: (i, 0)),
            pl.BlockSpec((8, D), lambda i: (0, 0)),
            pl.BlockSpec((1, D), lambda i: (0, 0)),
            pl.BlockSpec((1, D), lambda i: (0, 0)),
            pl.BlockSpec((NB, 1), lambda i: (i, 0)),
            pl.BlockSpec((D, H), lambda i: (0, 0)),
            pl.BlockSpec((1, H), lambda i: (0, 0)),
            pl.BlockSpec((H, 1), lambda i: (0, 0)),
            pl.BlockSpec((1, 1), lambda i: (0, 0)),
        ],
        out_specs=pl.BlockSpec((G, 1), lambda i: (0, 0)),
        out_shape=jax.ShapeDtypeStruct((G, 1), f32),
        scratch_shapes=[
            pltpu.VMEM((G, D), f32),
            pltpu.VMEM((G, 1), f32),
        ],
    )(h, stats, bng, bnb, batch2, ow1, ob1, ow2, ob2)


def kernel(x, edge_index, edge_attr, batch, emb,
           e1w0, e1b0, e2w0, e2b0, n1w0, n1b0, n2w0, n2b0, bng0, bnb0,
           e1w1, e1b1, e2w1, e2b1, n1w1, n1b1, n2w1, n2b1, bng1, bnb1,
           e1w2, e1b2, e2w2, e2b2, n1w2, n1b2, n2w2, n2b2, bng2, bnb2,
           ow1, ob1, ow2, ob2):
    f32 = jnp.float32
    x2 = x.astype(jnp.int32).reshape(N, 1)
    batch2 = batch.astype(jnp.int32).reshape(N, 1)
    src = edge_index[0].astype(jnp.int32)
    dst = edge_index[1].astype(jnp.int32)
    dst2 = dst.reshape(E, 1)

    e1w = [e1w0, e1w1, e1w2]
    e1b = [e1b0.reshape(1, H), e1b1.reshape(1, H), e1b2.reshape(1, H)]
    e2w = [e2w0, e2w1, e2w2]
    e2b = [e2b0.reshape(1, H), e2b1.reshape(1, H), e2b2.reshape(1, H)]
    n1w = [n1w0, n1w1, n1w2]
    n1b = [n1b0.reshape(1, H), n1b1.reshape(1, H), n1b2.reshape(1, H)]
    n2w = [n2w0, n2w1, n2w2]
    n2b = [n2b0.reshape(1, D), n2b1.reshape(1, D), n2b2.reshape(1, D)]
    bng = [bng0.reshape(1, D), bng1.reshape(1, D), bng2.reshape(1, D)]
    bnb = [bnb0.reshape(1, D), bnb1.reshape(1, D), bnb2.reshape(1, D)]

    stats = None
    h = None
    for l in range(3):
        wi = e1w[l][0:D]
        wj = e1w[l][D:2 * D]
        we = e1w[l][2 * D:2 * D + ED]
        wh = n1w[l][0:D]
        wa = n1w[l][D:2 * D]
        if l == 0:
            h, t1, t2, hh = _k0(x2, emb, wi, wj, wh, e1b[l], n1b[l])
            ab = jnp.concatenate([jnp.ones((1, H), f32), jnp.zeros((7, H), f32)], axis=0)
        else:
            t1, t2, hh, ab = _k5(h, stats, bng[l - 1], bnb[l - 1], wi, wj, wh,
                                 e1b[l], n1b[l])
        u = _k1(t1, t2, dst, src)
        ms = _k2(u, edge_attr, dst2, we, e2w[l], e2b[l])
        p0, p1 = _k3(ms, dst)
        aggr = jnp.concatenate([
            p0.reshape(PAD_PAIRS * 2, H)[0:N // 2],
            p1.reshape(PAD_PAIRS * 2, H)[0:N // 2],
        ], axis=0)
        h, stats = _k4(h, ab, hh, aggr, wa, n2w[l], n2b[l])

    return _k6(h, stats, bng[2], bnb[2], batch2, ow1, ob1.reshape(1, H),
               ow2, ob2.reshape(1, 1))


# speedup vs baseline: 1.6632x; 1.0464x over previous
import functools

import jax
import jax.numpy as jnp
from jax import lax
from jax.experimental import pallas as pl
from jax.experimental.pallas import tpu as pltpu
from jax.experimental.pallas import tpu_sc as plsc

N = 50000
E = 800000
D = 64
ED = 16
H = 64
G = 256

NB = 1000
N_STEPS = N // NB
EB = 6400
E_STEPS = E // EB

NW = 32
GCH = 128
NCHUNK = E // GCH
GSLAB = (NCHUNK // NW + 1) * GCH

PAIRS = N // 4
PAD_PAIRS = 12544
ZR = PAD_PAIRS // 16


def _softplus(x):
    return jnp.maximum(x, 0.0) + jnp.log(1.0 + jnp.exp(-jnp.abs(x)))


def _pq(h_eff, wi_ref, wj_ref, e1b_ref):
    a = jnp.dot(h_eff, wi_ref[...], preferred_element_type=jnp.float32, precision=lax.Precision.HIGHEST) + e1b_ref[...]
    b = jnp.dot(h_eff, wj_ref[...], preferred_element_type=jnp.float32, precision=lax.Precision.HIGHEST)
    par = (lax.broadcasted_iota(jnp.int32, (NB, H), 0) & 1).astype(jnp.float32)
    t1 = jnp.concatenate([a, par], axis=1)
    t2 = jnp.concatenate([b, jnp.zeros((NB, H), jnp.float32)], axis=1)
    return t1, t2


def _k0_body(x_ref, emb_ref, wi_ref, wj_ref, wh_ref, e1b_ref, n1b_ref,
             h_ref, t1_ref, t2_ref, hh_ref):
    xb = x_ref[...]
    oh = (xb == lax.broadcasted_iota(jnp.int32, (NB, 100), 1)).astype(jnp.float32)
    h = jnp.dot(oh, emb_ref[...], preferred_element_type=jnp.float32, precision=lax.Precision.HIGHEST)
    h_ref[...] = h
    t1_ref[...], t2_ref[...] = _pq(h, wi_ref, wj_ref, e1b_ref)
    hh_ref[...] = jnp.dot(h, wh_ref[...], preferred_element_type=jnp.float32, precision=lax.Precision.HIGHEST) + n1b_ref[...]


def _k0(x2, emb, wi, wj, wh, e1b, n1b):
    f32 = jnp.float32
    return pl.pallas_call(
        _k0_body,
        grid=(N_STEPS,),
        in_specs=[
            pl.BlockSpec((NB, 1), lambda i: (i, 0)),
            pl.BlockSpec((100, D), lambda i: (0, 0)),
            pl.BlockSpec((D, H), lambda i: (0, 0)),
            pl.BlockSpec((D, H), lambda i: (0, 0)),
            pl.BlockSpec((D, H), lambda i: (0, 0)),
            pl.BlockSpec((1, H), lambda i: (0, 0)),
            pl.BlockSpec((1, H), lambda i: (0, 0)),
        ],
        out_specs=[
            pl.BlockSpec((NB, D), lambda i: (i, 0)),
            pl.BlockSpec((NB, 2 * H), lambda i: (i, 0)),
            pl.BlockSpec((NB, 2 * H), lambda i: (i, 0)),
            pl.BlockSpec((NB, H), lambda i: (i, 0)),
        ],
        out_shape=[
            jax.ShapeDtypeStruct((N, D), f32),
            jax.ShapeDtypeStruct((N, 2 * H), f32),
            jax.ShapeDtypeStruct((N, 2 * H), f32),
            jax.ShapeDtypeStruct((N, H), f32),
        ],
    )(x2, emb, wi, wj, wh, e1b, n1b)


def _k5_body(h_ref, st_ref, bng_ref, bnb_ref, wi_ref, wj_ref, wh_ref,
             e1b_ref, n1b_ref, t1_ref, t2_ref, hh_ref, ab_ref):
    s = st_ref[0:1, :]
    sq = st_ref[1:2, :]
    mean = s * (1.0 / N)
    var = sq * (1.0 / N) - mean * mean
    alpha = bng_ref[...] * lax.rsqrt(var + 1e-5)
    beta = bnb_ref[...] - mean * alpha
    h_eff = h_ref[...] * alpha + beta
    t1_ref[...], t2_ref[...] = _pq(h_eff, wi_ref, wj_ref, e1b_ref)
    hh_ref[...] = jnp.dot(h_eff, wh_ref[...], preferred_element_type=jnp.float32, precision=lax.Precision.HIGHEST) + n1b_ref[...]
    ab_ref[...] = jnp.concatenate([alpha, beta, jnp.zeros((6, H), jnp.float32)], axis=0)


def _k5(h, stats, bng, bnb, wi, wj, wh, e1b, n1b):
    f32 = jnp.float32
    return pl.pallas_call(
        _k5_body,
        grid=(N_STEPS,),
        in_specs=[
            pl.BlockSpec((NB, D), lambda i: (i, 0)),
            pl.BlockSpec((8, H), lambda i: (0, 0)),
            pl.BlockSpec((1, H), lambda i: (0, 0)),
            pl.BlockSpec((1, H), lambda i: (0, 0)),
            pl.BlockSpec((D, H), lambda i: (0, 0)),
            pl.BlockSpec((D, H), lambda i: (0, 0)),
            pl.BlockSpec((D, H), lambda i: (0, 0)),
            pl.BlockSpec((1, H), lambda i: (0, 0)),
            pl.BlockSpec((1, H), lambda i: (0, 0)),
        ],
        out_specs=[
            pl.BlockSpec((NB, 2 * H), lambda i: (i, 0)),
            pl.BlockSpec((NB, 2 * H), lambda i: (i, 0)),
            pl.BlockSpec((NB, H), lambda i: (i, 0)),
            pl.BlockSpec((8, H), lambda i: (0, 0)),
        ],
        out_shape=[
            jax.ShapeDtypeStruct((N, 2 * H), f32),
            jax.ShapeDtypeStruct((N, 2 * H), f32),
            jax.ShapeDtypeStruct((N, H), f32),
            jax.ShapeDtypeStruct((8, H), f32),
        ],
    )(h, stats, bng, bnb, wi, wj, wh, e1b, n1b)


def _gather_body(t1_hbm, t2_hbm, dst_hbm, src_hbm, u_hbm, dslab, sslab, rows):
    c = lax.axis_index("c")
    s = lax.axis_index("s")
    wid = s * 2 + c
    cb = wid * NCHUNK // NW
    ce = (wid + 1) * NCHUNK // NW
    nch = ce - cb
    base = pl.multiple_of(cb * GCH, GCH)
    pltpu.sync_copy(dst_hbm.at[pl.ds(base, GSLAB - GCH)], dslab.at[pl.ds(0, GSLAB - GCH)])
    pltpu.sync_copy(src_hbm.at[pl.ds(base, GSLAB - GCH)], sslab.at[pl.ds(0, GSLAB - GCH)])

    @pl.when(nch * GCH == GSLAB)
    def _():
        off = pl.multiple_of(base + GSLAB - GCH, GCH)
        pltpu.sync_copy(dst_hbm.at[pl.ds(off, GCH)], dslab.at[pl.ds(GSLAB - GCH, GCH)])
        pltpu.sync_copy(src_hbm.at[pl.ds(off, GCH)], sslab.at[pl.ds(GSLAB - GCH, GCH)])

    def chunk(j, carry):
        off = pl.multiple_of(j * GCH, GCH)
        pltpu.sync_copy(t1_hbm.at[dslab.at[pl.ds(off, GCH)]], rows)
        pltpu.sync_copy(t2_hbm.at[sslab.at[pl.ds(off, GCH)]], rows, add=True)
        wb = pl.multiple_of(base + off, GCH)
        pltpu.sync_copy(rows, u_hbm.at[pl.ds(wb, GCH)])
        return carry

    lax.fori_loop(0, nch, chunk, 0)


def _k1(t1, t2, dst, src):
    mesh = plsc.VectorSubcoreMesh(core_axis_name="c", subcore_axis_name="s")
    fn = functools.partial(
        pl.kernel,
        out_type=jax.ShapeDtypeStruct((E, 2 * H), jnp.float32),
        mesh=mesh,
        scratch_types=[
            pltpu.VMEM((GSLAB,), jnp.int32),
            pltpu.VMEM((GSLAB,), jnp.int32),
            pltpu.VMEM((GCH, 2 * H), jnp.float32),
        ],
    )(_gather_body)
    return fn(t1, t2, dst, src)


def _k2_body(u_ref, eat_ref, we_ref, e2w_ref, e2b_ref, ms_ref):
    u = u_ref[...]
    c = lax.dot_general(eat_ref[...], we_ref[...], (((0,), (0,)), ((), ())),
                        preferred_element_type=jnp.float32,
                        precision=lax.Precision.HIGHEST)
    m = _softplus(u[:, 0:H] + c)
    m2 = _softplus(
        jnp.dot(m, e2w_ref[...], preferred_element_type=jnp.float32, precision=lax.Precision.HIGHEST) + e2b_ref[...])
    parf = u[:, H:H + 1]
    ms_ref[...] = jnp.concatenate([m2 * (1.0 - parf), m2 * parf], axis=1)


def _k2(u, eat, we, e2w, e2b):
    f32 = jnp.float32
    return pl.pallas_call(
        _k2_body,
        grid=(E_STEPS,),
        in_specs=[
            pl.BlockSpec((EB, 2 * H), lambda i: (i, 0)),
            pl.BlockSpec((ED, EB), lambda i: (0, i)),
            pl.BlockSpec((ED, H), lambda i: (0, 0)),
            pl.BlockSpec((H, H), lambda i: (0, 0)),
            pl.BlockSpec((1, H), lambda i: (0, 0)),
        ],
        out_specs=pl.BlockSpec((EB, 2 * H), lambda i: (i, 0)),
        out_shape=jax.ShapeDtypeStruct((E, 2 * H), f32),
    )(u, eat, we, e2w, e2b)


def _scatter_body(ms_hbm, dst_hbm, out0_hbm, out1_hbm,
                  aggr_sh, idxb, lidx, datab):
    c = lax.axis_index("c")
    s = lax.axis_index("s")

    def zr(r, carry):
        for g in range(8):
            datab[r, pl.ds(g * 16, 16)] = jnp.zeros((16,), jnp.float32)
        return carry
    lax.fori_loop(0, GCH, zr, 0)

    zbase = pl.multiple_of(s * ZR, 8)

    def zc(k, carry):
        pltpu.sync_copy(datab, aggr_sh.at[pl.ds(pl.multiple_of(zbase + k * GCH, 8), GCH)])
        return carry
    lax.fori_loop(0, ZR // GCH, zc, 0)
    zrem = ZR - (ZR // GCH) * GCH
    pltpu.sync_copy(datab.at[pl.ds(0, zrem)],
                    aggr_sh.at[pl.ds(pl.multiple_of(zbase + (ZR // GCH) * GCH, 8), zrem)])
    plsc.subcore_barrier()

    cb = s * NCHUNK // 16
    ce = (s + 1) * NCHUNK // 16
    lo = c * PAIRS

    def remap():
        for g in range(8):
            v = idxb[pl.ds(g * 16, 16)]
            local = lax.shift_right_logical(v, 1) - lo
            ok = (local >= 0) & (local < PAIRS)
            trash = PAIRS + (v & 7)
            lidx[pl.ds(g * 16, 16)] = jnp.where(ok, local, trash)

    def chunk(j, carry):
        eoff = pl.multiple_of(j * GCH, GCH)
        pltpu.sync_copy(dst_hbm.at[pl.ds(eoff, GCH)], idxb)
        remap()
        pltpu.sync_copy(ms_hbm.at[pl.ds(eoff, GCH)], datab)
        pltpu.sync_copy(datab, aggr_sh.at[lidx], add=True)
        return carry

    lax.fori_loop(cb, ce, chunk, 0)
    plsc.subcore_barrier()

    obase = pl.multiple_of(s * ZR, 8)

    @pl.when(c == 0)
    def _():
        pltpu.sync_copy(aggr_sh.at[pl.ds(obase, ZR)], out0_hbm.at[pl.ds(obase, ZR)])

    @pl.when(c == 1)
    def _():
        pltpu.sync_copy(aggr_sh.at[pl.ds(obase, ZR)], out1_hbm.at[pl.ds(obase, ZR)])


def _k3(ms, dst):
    f32 = jnp.float32
    mesh = plsc.VectorSubcoreMesh(core_axis_name="c", subcore_axis_name="s")
    fn = functools.partial(
        pl.kernel,
        out_type=[
            jax.ShapeDtypeStruct((PAD_PAIRS, 2 * H), f32),
            jax.ShapeDtypeStruct((PAD_PAIRS, 2 * H), f32),
        ],
        mesh=mesh,
        scratch_types=[
            pltpu.VMEM_SHARED((PAD_PAIRS, 2 * H), f32),
            pltpu.VMEM((GCH,), jnp.int32),
            pltpu.VMEM((GCH,), jnp.int32),
            pltpu.VMEM((GCH, 2 * H), f32),
        ],
    )(_scatter_body)
    return fn(ms, dst)


def _k4_body(h_ref, ab_ref, hh_ref, ag_ref, wa_ref, n2w_ref, n2b_ref,
             hn_ref, st_ref):
    i = pl.program_id(0)
    alpha = ab_ref[0:1, :]
    beta = ab_ref[1:2, :]
    h_eff = h_ref[...] * alpha + beta
    t = _softplus(hh_ref[...]
                  + jnp.dot(ag_ref[...], wa_ref[...], preferred_element_type=jnp.float32, precision=lax.Precision.HIGHEST))
    u = jnp.dot(t, n2w_ref[...], preferred_element_type=jnp.float32, precision=lax.Precision.HIGHEST) + n2b_ref[...]
    hn = u + h_eff
    hn_ref[...] = hn
    upd = jnp.concatenate([
        jnp.sum(hn, axis=0, keepdims=True),
        jnp.sum(hn * hn, axis=0, keepdims=True),
        jnp.zeros((6, H), jnp.float32),
    ], axis=0)

    @pl.when(i == 0)
    def _():
        st_ref[...] = upd

    @pl.when(i > 0)
    def _():
        st_ref[...] += upd


def _k4(h, ab, hh, aggr, wa, n2w, n2b):
    f32 = jnp.float32
    return pl.pallas_call(
        _k4_body,
        grid=(N_STEPS,),
        in_specs=[
            pl.BlockSpec((NB, D), lambda i: (i, 0)),
            pl.BlockSpec((8, H), lambda i: (0, 0)),
            pl.BlockSpec((NB, H), lambda i: (i, 0)),
            pl.BlockSpec((NB, H), lambda i: (i, 0)),
            pl.BlockSpec((H, H), lambda i: (0, 0)),
            pl.BlockSpec((H, D), lambda i: (0, 0)),
            pl.BlockSpec((1, D), lambda i: (0, 0)),
        ],
        out_specs=[
            pl.BlockSpec((NB, D), lambda i: (i, 0)),
            pl.BlockSpec((8, D), lambda i: (0, 0)),
        ],
        out_shape=[
            jax.ShapeDtypeStruct((N, D), f32),
            jax.ShapeDtypeStruct((8, D), f32),
        ],
    )(h, ab, hh, aggr, wa, n2w, n2b)


def _k6_body(h_ref, st_ref, bng_ref, bnb_ref, b_ref, ow1_ref, ob1_ref,
             ow2_ref, ob2_ref, out_ref, sums, counts):
    i = pl.program_id(0)
    s = st_ref[0:1, :]
    sq = st_ref[1:2, :]
    mean = s * (1.0 / N)
    var = sq * (1.0 / N) - mean * mean
    alpha = bng_ref[...] * lax.rsqrt(var + 1e-5)
    beta = bnb_ref[...] - mean * alpha
    h_eff = h_ref[...] * alpha + beta
    bb = b_ref[...]
    oh = (bb == lax.broadcasted_iota(jnp.int32, (NB, G), 1)).astype(jnp.float32)
    dnums = (((0,), (0,)), ((), ()))
    sum_c = lax.dot_general(oh, h_eff, dnums, preferred_element_type=jnp.float32, precision=lax.Precision.HIGHEST)
    cnt_c = lax.dot_general(oh, jnp.ones((NB, 1), jnp.float32), dnums,
                            preferred_element_type=jnp.float32, precision=lax.Precision.HIGHEST)

    @pl.when(i == 0)
    def _():
        sums[...] = sum_c
        counts[...] = cnt_c

    @pl.when(i > 0)
    def _():
        sums[...] += sum_c
        counts[...] += cnt_c

    @pl.when(i == N_STEPS - 1)
    def _():
        pooled = sums[...] / jnp.maximum(counts[...], 1.0)
        o1 = _softplus(jnp.dot(pooled, ow1_ref[...], preferred_element_type=jnp.float32, precision=lax.Precision.HIGHEST)
                       + ob1_ref[...])
        out_ref[...] = jnp.dot(o1, ow2_ref[...], preferred_element_type=jnp.float32, precision=lax.Precision.HIGHEST) + ob2_ref[...]


def _k6(h, stats, bng, bnb, batch2, ow1, ob1, ow2, ob2):
    f32 = jnp.float32
    return pl.pallas_call(
        _k6_body,
        grid=(N_STEPS,),
        in_specs=[
            pl.BlockSpec((NB, D), lambda i: (i, 0)),
            pl.BlockSpec((8, D), lambda i: (0, 0)),
            pl.BlockSpec((1, D), lambda i: (0, 0)),
            pl.BlockSpec((1, D), lambda i: (0, 0)),
            pl.BlockSpec((NB, 1), lambda i: (i, 0)),
            pl.BlockSpec((D, H), lambda i: (0, 0)),
            pl.BlockSpec((1, H), lambda i: (0, 0)),
            pl.BlockSpec((H, 1), lambda i: (0, 0)),
            pl.BlockSpec((1, 1), lambda i: (0, 0)),
        ],
        out_specs=pl.BlockSpec((G, 1), lambda i: (0, 0)),
        out_shape=jax.ShapeDtypeStruct((G, 1), f32),
        scratch_shapes=[
            pltpu.VMEM((G, D), f32),
            pltpu.VMEM((G, 1), f32),
        ],
    )(h, stats, bng, bnb, batch2, ow1, ob1, ow2, ob2)


def kernel(x, edge_index, edge_attr, batch, emb,
           e1w0, e1b0, e2w0, e2b0, n1w0, n1b0, n2w0, n2b0, bng0, bnb0,
           e1w1, e1b1, e2w1, e2b1, n1w1, n1b1, n2w1, n2b1, bng1, bnb1,
           e1w2, e1b2, e2w2, e2b2, n1w2, n1b2, n2w2, n2b2, bng2, bnb2,
           ow1, ob1, ow2, ob2):
    f32 = jnp.float32
    x2 = x.astype(jnp.int32).reshape(N, 1)
    batch2 = batch.astype(jnp.int32).reshape(N, 1)
    src = edge_index[0].astype(jnp.int32)
    dst = edge_index[1].astype(jnp.int32)
    eat = edge_attr.T

    e1w = [e1w0, e1w1, e1w2]
    e1b = [e1b0.reshape(1, H), e1b1.reshape(1, H), e1b2.reshape(1, H)]
    e2w = [e2w0, e2w1, e2w2]
    e2b = [e2b0.reshape(1, H), e2b1.reshape(1, H), e2b2.reshape(1, H)]
    n1w = [n1w0, n1w1, n1w2]
    n1b = [n1b0.reshape(1, H), n1b1.reshape(1, H), n1b2.reshape(1, H)]
    n2w = [n2w0, n2w1, n2w2]
    n2b = [n2b0.reshape(1, D), n2b1.reshape(1, D), n2b2.reshape(1, D)]
    bng = [bng0.reshape(1, D), bng1.reshape(1, D), bng2.reshape(1, D)]
    bnb = [bnb0.reshape(1, D), bnb1.reshape(1, D), bnb2.reshape(1, D)]

    stats = None
    h = None
    for l in range(3):
        wi = e1w[l][0:D]
        wj = e1w[l][D:2 * D]
        we = e1w[l][2 * D:2 * D + ED]
        wh = n1w[l][0:D]
        wa = n1w[l][D:2 * D]
        if l == 0:
            h, t1, t2, hh = _k0(x2, emb, wi, wj, wh, e1b[l], n1b[l])
            ab = jnp.concatenate([jnp.ones((1, H), f32), jnp.zeros((7, H), f32)], axis=0)
        else:
            t1, t2, hh, ab = _k5(h, stats, bng[l - 1], bnb[l - 1], wi, wj, wh,
                                 e1b[l], n1b[l])
        u = _k1(t1, t2, dst, src)
        ms = _k2(u, eat, we, e2w[l], e2b[l])
        p0, p1 = _k3(ms, dst)
        aggr = jnp.concatenate([
            p0.reshape(PAD_PAIRS * 2, H)[0:N // 2],
            p1.reshape(PAD_PAIRS * 2, H)[0:N // 2],
        ], axis=0)
        h, stats = _k4(h, ab, hh, aggr, wa, n2w[l], n2b[l])

    return _k6(h, stats, bng[2], bnb[2], batch2, ow1, ob1.reshape(1, H),
               ow2, ob2.reshape(1, 1))


# speedup vs baseline: 2.1292x; 1.2802x over previous
import functools

import jax
import jax.numpy as jnp
from jax import lax
from jax.experimental import pallas as pl
from jax.experimental.pallas import tpu as pltpu
from jax.experimental.pallas import tpu_sc as plsc

N = 50000
E = 800000
D = 64
ED = 16
H = 64
G = 256

NB = 1000
N_STEPS = N // NB
EB = 6400
E_STEPS = E // EB

NW = 32
GCH = 128
NCHUNK = E // GCH
GSLAB = (NCHUNK // NW + 1) * GCH

PAIRS = N // 4
PAD_PAIRS = 12544
ZR = PAD_PAIRS // 16


def _softplus(x):
    return jnp.maximum(x, 0.0) + jnp.log(1.0 + jnp.exp(-jnp.abs(x)))


def _pq(h_eff, wi_ref, wj_ref, e1b_ref):
    a = jnp.dot(h_eff, wi_ref[...], preferred_element_type=jnp.float32, precision=lax.Precision.HIGHEST) + e1b_ref[...]
    b = jnp.dot(h_eff, wj_ref[...], preferred_element_type=jnp.float32, precision=lax.Precision.HIGHEST)
    par = (lax.broadcasted_iota(jnp.int32, (NB, H), 0) & 1).astype(jnp.float32)
    t1 = jnp.concatenate([a, par], axis=1)
    t2 = jnp.concatenate([b, jnp.zeros((NB, H), jnp.float32)], axis=1)
    return t1, t2


def _k0_body(x_ref, emb_ref, wi_ref, wj_ref, wh_ref, e1b_ref, n1b_ref,
             h_ref, t1_ref, t2_ref, hh_ref):
    xb = x_ref[...]
    oh = (xb == lax.broadcasted_iota(jnp.int32, (NB, 100), 1)).astype(jnp.float32)
    h = jnp.dot(oh, emb_ref[...], preferred_element_type=jnp.float32, precision=lax.Precision.HIGHEST)
    h_ref[...] = h
    t1_ref[...], t2_ref[...] = _pq(h, wi_ref, wj_ref, e1b_ref)
    hh_ref[...] = jnp.dot(h, wh_ref[...], preferred_element_type=jnp.float32, precision=lax.Precision.HIGHEST) + n1b_ref[...]


def _k0(x2, emb, wi, wj, wh, e1b, n1b):
    f32 = jnp.float32
    return pl.pallas_call(
        _k0_body,
        grid=(N_STEPS,),
        in_specs=[
            pl.BlockSpec((NB, 1), lambda i: (i, 0)),
            pl.BlockSpec((100, D), lambda i: (0, 0)),
            pl.BlockSpec((D, H), lambda i: (0, 0)),
            pl.BlockSpec((D, H), lambda i: (0, 0)),
            pl.BlockSpec((D, H), lambda i: (0, 0)),
            pl.BlockSpec((1, H), lambda i: (0, 0)),
            pl.BlockSpec((1, H), lambda i: (0, 0)),
        ],
        out_specs=[
            pl.BlockSpec((NB, D), lambda i: (i, 0)),
            pl.BlockSpec((NB, 2 * H), lambda i: (i, 0)),
            pl.BlockSpec((NB, 2 * H), lambda i: (i, 0)),
            pl.BlockSpec((NB, H), lambda i: (i, 0)),
        ],
        out_shape=[
            jax.ShapeDtypeStruct((N, D), f32),
            jax.ShapeDtypeStruct((N, 2 * H), f32),
            jax.ShapeDtypeStruct((N, 2 * H), f32),
            jax.ShapeDtypeStruct((N, H), f32),
        ],
    )(x2, emb, wi, wj, wh, e1b, n1b)


def _k5_body(h_ref, st_ref, bng_ref, bnb_ref, wi_ref, wj_ref, wh_ref,
             e1b_ref, n1b_ref, t1_ref, t2_ref, hh_ref, ab_ref):
    s = st_ref[0:1, :]
    sq = st_ref[1:2, :]
    mean = s * (1.0 / N)
    var = sq * (1.0 / N) - mean * mean
    alpha = bng_ref[...] * lax.rsqrt(var + 1e-5)
    beta = bnb_ref[...] - mean * alpha
    h_eff = h_ref[...] * alpha + beta
    t1_ref[...], t2_ref[...] = _pq(h_eff, wi_ref, wj_ref, e1b_ref)
    hh_ref[...] = jnp.dot(h_eff, wh_ref[...], preferred_element_type=jnp.float32, precision=lax.Precision.HIGHEST) + n1b_ref[...]
    ab_ref[...] = jnp.concatenate([alpha, beta, jnp.zeros((6, H), jnp.float32)], axis=0)


def _k5(h, stats, bng, bnb, wi, wj, wh, e1b, n1b):
    f32 = jnp.float32
    return pl.pallas_call(
        _k5_body,
        grid=(N_STEPS,),
        in_specs=[
            pl.BlockSpec((NB, D), lambda i: (i, 0)),
            pl.BlockSpec((8, H), lambda i: (0, 0)),
            pl.BlockSpec((1, H), lambda i: (0, 0)),
            pl.BlockSpec((1, H), lambda i: (0, 0)),
            pl.BlockSpec((D, H), lambda i: (0, 0)),
            pl.BlockSpec((D, H), lambda i: (0, 0)),
            pl.BlockSpec((D, H), lambda i: (0, 0)),
            pl.BlockSpec((1, H), lambda i: (0, 0)),
            pl.BlockSpec((1, H), lambda i: (0, 0)),
        ],
        out_specs=[
            pl.BlockSpec((NB, 2 * H), lambda i: (i, 0)),
            pl.BlockSpec((NB, 2 * H), lambda i: (i, 0)),
            pl.BlockSpec((NB, H), lambda i: (i, 0)),
            pl.BlockSpec((8, H), lambda i: (0, 0)),
        ],
        out_shape=[
            jax.ShapeDtypeStruct((N, 2 * H), f32),
            jax.ShapeDtypeStruct((N, 2 * H), f32),
            jax.ShapeDtypeStruct((N, H), f32),
            jax.ShapeDtypeStruct((8, H), f32),
        ],
    )(h, stats, bng, bnb, wi, wj, wh, e1b, n1b)


def _gather_body(t1_hbm, t2_hbm, dst_hbm, src_hbm, u_hbm, dslab, sslab, rows):
    c = lax.axis_index("c")
    s = lax.axis_index("s")
    wid = s * 2 + c
    cb = wid * NCHUNK // NW
    ce = (wid + 1) * NCHUNK // NW
    nch = ce - cb
    base = pl.multiple_of(cb * GCH, GCH)
    pltpu.sync_copy(dst_hbm.at[pl.ds(base, GSLAB - GCH)], dslab.at[pl.ds(0, GSLAB - GCH)])
    pltpu.sync_copy(src_hbm.at[pl.ds(base, GSLAB - GCH)], sslab.at[pl.ds(0, GSLAB - GCH)])

    @pl.when(nch * GCH == GSLAB)
    def _():
        off = pl.multiple_of(base + GSLAB - GCH, GCH)
        pltpu.sync_copy(dst_hbm.at[pl.ds(off, GCH)], dslab.at[pl.ds(GSLAB - GCH, GCH)])
        pltpu.sync_copy(src_hbm.at[pl.ds(off, GCH)], sslab.at[pl.ds(GSLAB - GCH, GCH)])

    def chunk(j, carry):
        off = pl.multiple_of(j * GCH, GCH)
        pltpu.sync_copy(t1_hbm.at[dslab.at[pl.ds(off, GCH)]], rows)
        pltpu.sync_copy(t2_hbm.at[sslab.at[pl.ds(off, GCH)]], rows, add=True)
        wb = pl.multiple_of(base + off, GCH)
        pltpu.sync_copy(rows, u_hbm.at[pl.ds(wb, GCH)])
        return carry

    lax.fori_loop(0, nch, chunk, 0)


def _k1(t1, t2, dst, src):
    mesh = plsc.VectorSubcoreMesh(core_axis_name="c", subcore_axis_name="s")
    fn = functools.partial(
        pl.kernel,
        out_type=jax.ShapeDtypeStruct((E, 2 * H), jnp.float32),
        mesh=mesh,
        scratch_types=[
            pltpu.VMEM((GSLAB,), jnp.int32),
            pltpu.VMEM((GSLAB,), jnp.int32),
            pltpu.VMEM((GCH, 2 * H), jnp.float32),
        ],
    )(_gather_body)
    return fn(t1, t2, dst, src)


def _k2_body(u_ref, eat_ref, we_ref, e2w_ref, e2b_ref, ms_ref):
    u = u_ref[...]
    c = lax.dot_general(eat_ref[...], we_ref[...], (((0,), (0,)), ((), ())),
                        preferred_element_type=jnp.float32,
                        precision=lax.Precision.HIGHEST)
    m = _softplus(u[:, 0:H] + c)
    m2 = _softplus(
        jnp.dot(m, e2w_ref[...], preferred_element_type=jnp.float32) + e2b_ref[...])
    parf = u[:, H:H + 1]
    ms_ref[...] = jnp.concatenate([m2 * (1.0 - parf), m2 * parf], axis=1)


def _k2(u, eat, we, e2w, e2b):
    f32 = jnp.float32
    return pl.pallas_call(
        _k2_body,
        grid=(E_STEPS,),
        in_specs=[
            pl.BlockSpec((EB, 2 * H), lambda i: (i, 0)),
            pl.BlockSpec((ED, EB), lambda i: (0, i)),
            pl.BlockSpec((ED, H), lambda i: (0, 0)),
            pl.BlockSpec((H, H), lambda i: (0, 0)),
            pl.BlockSpec((1, H), lambda i: (0, 0)),
        ],
        out_specs=pl.BlockSpec((EB, 2 * H), lambda i: (i, 0)),
        out_shape=jax.ShapeDtypeStruct((E, 2 * H), f32),
    )(u, eat, we, e2w, e2b)


def _scatter_body(ms_hbm, dst_hbm, out0_hbm, out1_hbm,
                  aggr_sh, idxb, lidx, datab):
    c = lax.axis_index("c")
    s = lax.axis_index("s")

    def zr(r, carry):
        for g in range(8):
            datab[r, pl.ds(g * 16, 16)] = jnp.zeros((16,), jnp.float32)
        return carry
    lax.fori_loop(0, GCH, zr, 0)

    zbase = pl.multiple_of(s * ZR, 8)

    def zc(k, carry):
        pltpu.sync_copy(datab, aggr_sh.at[pl.ds(pl.multiple_of(zbase + k * GCH, 8), GCH)])
        return carry
    lax.fori_loop(0, ZR // GCH, zc, 0)
    zrem = ZR - (ZR // GCH) * GCH
    pltpu.sync_copy(datab.at[pl.ds(0, zrem)],
                    aggr_sh.at[pl.ds(pl.multiple_of(zbase + (ZR // GCH) * GCH, 8), zrem)])
    plsc.subcore_barrier()

    cb = s * NCHUNK // 16
    ce = (s + 1) * NCHUNK // 16
    lo = c * PAIRS

    def remap():
        for g in range(8):
            v = idxb[pl.ds(g * 16, 16)]
            local = lax.shift_right_logical(v, 1) - lo
            ok = (local >= 0) & (local < PAIRS)
            trash = PAIRS + (v & 7)
            lidx[pl.ds(g * 16, 16)] = jnp.where(ok, local, trash)

    def chunk(j, carry):
        eoff = pl.multiple_of(j * GCH, GCH)
        pltpu.sync_copy(dst_hbm.at[pl.ds(eoff, GCH)], idxb)
        remap()
        pltpu.sync_copy(ms_hbm.at[pl.ds(eoff, GCH)], datab)
        pltpu.sync_copy(datab, aggr_sh.at[lidx], add=True)
        return carry

    lax.fori_loop(cb, ce, chunk, 0)
    plsc.subcore_barrier()

    obase = pl.multiple_of(s * ZR, 8)

    @pl.when(c == 0)
    def _():
        pltpu.sync_copy(aggr_sh.at[pl.ds(obase, ZR)], out0_hbm.at[pl.ds(obase, ZR)])

    @pl.when(c == 1)
    def _():
        pltpu.sync_copy(aggr_sh.at[pl.ds(obase, ZR)], out1_hbm.at[pl.ds(obase, ZR)])


def _k3(ms, dst):
    f32 = jnp.float32
    mesh = plsc.VectorSubcoreMesh(core_axis_name="c", subcore_axis_name="s")
    fn = functools.partial(
        pl.kernel,
        out_type=[
            jax.ShapeDtypeStruct((PAD_PAIRS, 2 * H), f32),
            jax.ShapeDtypeStruct((PAD_PAIRS, 2 * H), f32),
        ],
        mesh=mesh,
        scratch_types=[
            pltpu.VMEM_SHARED((PAD_PAIRS, 2 * H), f32),
            pltpu.VMEM((GCH,), jnp.int32),
            pltpu.VMEM((GCH,), jnp.int32),
            pltpu.VMEM((GCH, 2 * H), f32),
        ],
    )(_scatter_body)
    return fn(ms, dst)


def _k4_body(h_ref, ab_ref, hh_ref, ag_ref, wa_ref, n2w_ref, n2b_ref,
             hn_ref, st_ref):
    i = pl.program_id(0)
    alpha = ab_ref[0:1, :]
    beta = ab_ref[1:2, :]
    h_eff = h_ref[...] * alpha + beta
    t = _softplus(hh_ref[...]
                  + jnp.dot(ag_ref[...], wa_ref[...], preferred_element_type=jnp.float32, precision=lax.Precision.HIGHEST))
    u = jnp.dot(t, n2w_ref[...], preferred_element_type=jnp.float32, precision=lax.Precision.HIGHEST) + n2b_ref[...]
    hn = u + h_eff
    hn_ref[...] = hn
    upd = jnp.concatenate([
        jnp.sum(hn, axis=0, keepdims=True),
        jnp.sum(hn * hn, axis=0, keepdims=True),
        jnp.zeros((6, H), jnp.float32),
    ], axis=0)

    @pl.when(i == 0)
    def _():
        st_ref[...] = upd

    @pl.when(i > 0)
    def _():
        st_ref[...] += upd


def _k4(h, ab, hh, aggr, wa, n2w, n2b):
    f32 = jnp.float32
    return pl.pallas_call(
        _k4_body,
        grid=(N_STEPS,),
        in_specs=[
            pl.BlockSpec((NB, D), lambda i: (i, 0)),
            pl.BlockSpec((8, H), lambda i: (0, 0)),
            pl.BlockSpec((NB, H), lambda i: (i, 0)),
            pl.BlockSpec((NB, H), lambda i: (i, 0)),
            pl.BlockSpec((H, H), lambda i: (0, 0)),
            pl.BlockSpec((H, D), lambda i: (0, 0)),
            pl.BlockSpec((1, D), lambda i: (0, 0)),
        ],
        out_specs=[
            pl.BlockSpec((NB, D), lambda i: (i, 0)),
            pl.BlockSpec((8, D), lambda i: (0, 0)),
        ],
        out_shape=[
            jax.ShapeDtypeStruct((N, D), f32),
            jax.ShapeDtypeStruct((8, D), f32),
        ],
    )(h, ab, hh, aggr, wa, n2w, n2b)


def _k6_body(h_ref, st_ref, bng_ref, bnb_ref, b_ref, ow1_ref, ob1_ref,
             ow2_ref, ob2_ref, out_ref, sums, counts):
    i = pl.program_id(0)
    s = st_ref[0:1, :]
    sq = st_ref[1:2, :]
    mean = s * (1.0 / N)
    var = sq * (1.0 / N) - mean * mean
    alpha = bng_ref[...] * lax.rsqrt(var + 1e-5)
    beta = bnb_ref[...] - mean * alpha
    h_eff = h_ref[...] * alpha + beta
    bb = b_ref[...]
    oh = (bb == lax.broadcasted_iota(jnp.int32, (NB, G), 1)).astype(jnp.float32)
    dnums = (((0,), (0,)), ((), ()))
    sum_c = lax.dot_general(oh, h_eff, dnums, preferred_element_type=jnp.float32, precision=lax.Precision.HIGHEST)
    cnt_c = lax.dot_general(oh, jnp.ones((NB, 1), jnp.float32), dnums,
                            preferred_element_type=jnp.float32, precision=lax.Precision.HIGHEST)

    @pl.when(i == 0)
    def _():
        sums[...] = sum_c
        counts[...] = cnt_c

    @pl.when(i > 0)
    def _():
        sums[...] += sum_c
        counts[...] += cnt_c

    @pl.when(i == N_STEPS - 1)
    def _():
        pooled = sums[...] / jnp.maximum(counts[...], 1.0)
        o1 = _softplus(jnp.dot(pooled, ow1_ref[...], preferred_element_type=jnp.float32, precision=lax.Precision.HIGHEST)
                       + ob1_ref[...])
        out_ref[...] = jnp.dot(o1, ow2_ref[...], preferred_element_type=jnp.float32, precision=lax.Precision.HIGHEST) + ob2_ref[...]


def _k6(h, stats, bng, bnb, batch2, ow1, ob1, ow2, ob2):
    f32 = jnp.float32
    return pl.pallas_call(
        _k6_body,
        grid=(N_STEPS,),
        in_specs=[
            pl.BlockSpec((NB, D), lambda i: (i, 0)),
            pl.BlockSpec((8, D), lambda i: (0, 0)),
            pl.BlockSpec((1, D), lambda i: (0, 0)),
            pl.BlockSpec((1, D), lambda i: (0, 0)),
            pl.BlockSpec((NB, 1), lambda i: (i, 0)),
            pl.BlockSpec((D, H), lambda i: (0, 0)),
            pl.BlockSpec((1, H), lambda i: (0, 0)),
            pl.BlockSpec((H, 1), lambda i: (0, 0)),
            pl.BlockSpec((1, 1), lambda i: (0, 0)),
        ],
        out_specs=pl.BlockSpec((G, 1), lambda i: (0, 0)),
        out_shape=jax.ShapeDtypeStruct((G, 1), f32),
        scratch_shapes=[
            pltpu.VMEM((G, D), f32),
            pltpu.VMEM((G, 1), f32),
        ],
    )(h, stats, bng, bnb, batch2, ow1, ob1, ow2, ob2)


def kernel(x, edge_index, edge_attr, batch, emb,
           e1w0, e1b0, e2w0, e2b0, n1w0, n1b0, n2w0, n2b0, bng0, bnb0,
           e1w1, e1b1, e2w1, e2b1, n1w1, n1b1, n2w1, n2b1, bng1, bnb1,
           e1w2, e1b2, e2w2, e2b2, n1w2, n1b2, n2w2, n2b2, bng2, bnb2,
           ow1, ob1, ow2, ob2):
    f32 = jnp.float32
    x2 = x.astype(jnp.int32).reshape(N, 1)
    batch2 = batch.astype(jnp.int32).reshape(N, 1)
    src = edge_index[0].astype(jnp.int32)
    dst = edge_index[1].astype(jnp.int32)
    eat = edge_attr.T

    e1w = [e1w0, e1w1, e1w2]
    e1b = [e1b0.reshape(1, H), e1b1.reshape(1, H), e1b2.reshape(1, H)]
    e2w = [e2w0, e2w1, e2w2]
    e2b = [e2b0.reshape(1, H), e2b1.reshape(1, H), e2b2.reshape(1, H)]
    n1w = [n1w0, n1w1, n1w2]
    n1b = [n1b0.reshape(1, H), n1b1.reshape(1, H), n1b2.reshape(1, H)]
    n2w = [n2w0, n2w1, n2w2]
    n2b = [n2b0.reshape(1, D), n2b1.reshape(1, D), n2b2.reshape(1, D)]
    bng = [bng0.reshape(1, D), bng1.reshape(1, D), bng2.reshape(1, D)]
    bnb = [bnb0.reshape(1, D), bnb1.reshape(1, D), bnb2.reshape(1, D)]

    stats = None
    h = None
    for l in range(3):
        wi = e1w[l][0:D]
        wj = e1w[l][D:2 * D]
        we = e1w[l][2 * D:2 * D + ED]
        wh = n1w[l][0:D]
        wa = n1w[l][D:2 * D]
        if l == 0:
            h, t1, t2, hh = _k0(x2, emb, wi, wj, wh, e1b[l], n1b[l])
            ab = jnp.concatenate([jnp.ones((1, H), f32), jnp.zeros((7, H), f32)], axis=0)
        else:
            t1, t2, hh, ab = _k5(h, stats, bng[l - 1], bnb[l - 1], wi, wj, wh,
                                 e1b[l], n1b[l])
        u = _k1(t1, t2, dst, src)
        ms = _k2(u, eat, we, e2w[l], e2b[l])
        p0, p1 = _k3(ms, dst)
        aggr = jnp.concatenate([
            p0.reshape(PAD_PAIRS * 2, H)[0:N // 2],
            p1.reshape(PAD_PAIRS * 2, H)[0:N // 2],
        ], axis=0)
        h, stats = _k4(h, ab, hh, aggr, wa, n2w[l], n2b[l])

    return _k6(h, stats, bng[2], bnb[2], batch2, ow1, ob1.reshape(1, H),
               ow2, ob2.reshape(1, 1))


# speedup vs baseline: 2.3108x; 1.0853x over previous
import functools

import jax
import jax.numpy as jnp
from jax import lax
from jax.experimental import pallas as pl
from jax.experimental.pallas import tpu as pltpu
from jax.experimental.pallas import tpu_sc as plsc

N = 50000
E = 800000
D = 64
ED = 16
H = 64
G = 256

NB = 1000
N_STEPS = N // NB
EB = 6400
E_STEPS = E // EB

NW = 32
GCH = 128
NCHUNK = E // GCH
GSLAB = (NCHUNK // NW + 1) * GCH

PAIRS = N // 4
PAD_PAIRS = 12544
ZR = PAD_PAIRS // 16


def _softplus(x):
    return jnp.maximum(x, 0.0) + jnp.log(1.0 + jnp.exp(-jnp.abs(x)))


def _pq(h_eff, wi_ref, wj_ref, e1b_ref):
    a = jnp.dot(h_eff, wi_ref[...], preferred_element_type=jnp.float32, precision=lax.Precision.HIGHEST) + e1b_ref[...]
    b = jnp.dot(h_eff, wj_ref[...], preferred_element_type=jnp.float32, precision=lax.Precision.HIGHEST)
    par = (lax.broadcasted_iota(jnp.int32, (NB, H), 0) & 1).astype(jnp.float32)
    t1 = jnp.concatenate([a, par], axis=1)
    t2 = jnp.concatenate([b, jnp.zeros((NB, H), jnp.float32)], axis=1)
    return t1, t2


def _k0_body(x_ref, emb_ref, wi_ref, wj_ref, wh_ref, e1b_ref, n1b_ref,
             h_ref, t1_ref, t2_ref, hh_ref):
    xb = x_ref[...]
    oh = (xb == lax.broadcasted_iota(jnp.int32, (NB, 100), 1)).astype(jnp.float32)
    h = jnp.dot(oh, emb_ref[...], preferred_element_type=jnp.float32, precision=lax.Precision.HIGHEST)
    h_ref[...] = h
    t1_ref[...], t2_ref[...] = _pq(h, wi_ref, wj_ref, e1b_ref)
    hh_ref[...] = jnp.dot(h, wh_ref[...], preferred_element_type=jnp.float32, precision=lax.Precision.HIGHEST) + n1b_ref[...]


def _k0(x2, emb, wi, wj, wh, e1b, n1b):
    f32 = jnp.float32
    return pl.pallas_call(
        _k0_body,
        grid=(N_STEPS,),
        in_specs=[
            pl.BlockSpec((NB, 1), lambda i: (i, 0)),
            pl.BlockSpec((100, D), lambda i: (0, 0)),
            pl.BlockSpec((D, H), lambda i: (0, 0)),
            pl.BlockSpec((D, H), lambda i: (0, 0)),
            pl.BlockSpec((D, H), lambda i: (0, 0)),
            pl.BlockSpec((1, H), lambda i: (0, 0)),
            pl.BlockSpec((1, H), lambda i: (0, 0)),
        ],
        out_specs=[
            pl.BlockSpec((NB, D), lambda i: (i, 0)),
            pl.BlockSpec((NB, 2 * H), lambda i: (i, 0)),
            pl.BlockSpec((NB, 2 * H), lambda i: (i, 0)),
            pl.BlockSpec((NB, H), lambda i: (i, 0)),
        ],
        out_shape=[
            jax.ShapeDtypeStruct((N, D), f32),
            jax.ShapeDtypeStruct((N, 2 * H), f32),
            jax.ShapeDtypeStruct((N, 2 * H), f32),
            jax.ShapeDtypeStruct((N, H), f32),
        ],
    )(x2, emb, wi, wj, wh, e1b, n1b)


def _k5_body(h_ref, st_ref, bng_ref, bnb_ref, wi_ref, wj_ref, wh_ref,
             e1b_ref, n1b_ref, t1_ref, t2_ref, hh_ref, ab_ref):
    s = st_ref[0:1, :]
    sq = st_ref[1:2, :]
    mean = s * (1.0 / N)
    var = sq * (1.0 / N) - mean * mean
    alpha = bng_ref[...] * lax.rsqrt(var + 1e-5)
    beta = bnb_ref[...] - mean * alpha
    h_eff = h_ref[...] * alpha + beta
    t1_ref[...], t2_ref[...] = _pq(h_eff, wi_ref, wj_ref, e1b_ref)
    hh_ref[...] = jnp.dot(h_eff, wh_ref[...], preferred_element_type=jnp.float32, precision=lax.Precision.HIGHEST) + n1b_ref[...]
    ab_ref[...] = jnp.concatenate([alpha, beta, jnp.zeros((6, H), jnp.float32)], axis=0)


def _k5(h, stats, bng, bnb, wi, wj, wh, e1b, n1b):
    f32 = jnp.float32
    return pl.pallas_call(
        _k5_body,
        grid=(N_STEPS,),
        in_specs=[
            pl.BlockSpec((NB, D), lambda i: (i, 0)),
            pl.BlockSpec((8, H), lambda i: (0, 0)),
            pl.BlockSpec((1, H), lambda i: (0, 0)),
            pl.BlockSpec((1, H), lambda i: (0, 0)),
            pl.BlockSpec((D, H), lambda i: (0, 0)),
            pl.BlockSpec((D, H), lambda i: (0, 0)),
            pl.BlockSpec((D, H), lambda i: (0, 0)),
            pl.BlockSpec((1, H), lambda i: (0, 0)),
            pl.BlockSpec((1, H), lambda i: (0, 0)),
        ],
        out_specs=[
            pl.BlockSpec((NB, 2 * H), lambda i: (i, 0)),
            pl.BlockSpec((NB, 2 * H), lambda i: (i, 0)),
            pl.BlockSpec((NB, H), lambda i: (i, 0)),
            pl.BlockSpec((8, H), lambda i: (0, 0)),
        ],
        out_shape=[
            jax.ShapeDtypeStruct((N, 2 * H), f32),
            jax.ShapeDtypeStruct((N, 2 * H), f32),
            jax.ShapeDtypeStruct((N, H), f32),
            jax.ShapeDtypeStruct((8, H), f32),
        ],
    )(h, stats, bng, bnb, wi, wj, wh, e1b, n1b)


def _gather_body(t1_hbm, t2_hbm, dst_hbm, src_hbm, u_hbm, dslab, sslab,
                 rows0, rows1, rows2, semg, semw):
    c = lax.axis_index("c")
    s = lax.axis_index("s")
    wid = s * 2 + c
    cb = wid * NCHUNK // NW
    ce = (wid + 1) * NCHUNK // NW
    nch = ce - cb
    base = pl.multiple_of(cb * GCH, GCH)
    pltpu.sync_copy(dst_hbm.at[pl.ds(base, GSLAB - GCH)], dslab.at[pl.ds(0, GSLAB - GCH)])
    pltpu.sync_copy(src_hbm.at[pl.ds(base, GSLAB - GCH)], sslab.at[pl.ds(0, GSLAB - GCH)])

    @pl.when(nch * GCH == GSLAB)
    def _():
        off = pl.multiple_of(base + GSLAB - GCH, GCH)
        pltpu.sync_copy(dst_hbm.at[pl.ds(off, GCH)], dslab.at[pl.ds(GSLAB - GCH, GCH)])
        pltpu.sync_copy(src_hbm.at[pl.ds(off, GCH)], sslab.at[pl.ds(GSLAB - GCH, GCH)])

    rows = (rows0, rows1, rows2)

    def grp(g, carry):
        @pl.when(g > 0)
        def _():
            for b in range(3):
                pltpu.make_async_copy(rows[b], u_hbm.at[pl.ds(0, GCH)], semw).wait()

        offs = [pl.multiple_of((g * 3 + b) * GCH, GCH) for b in range(3)]
        d1 = [pltpu.async_copy(t1_hbm.at[dslab.at[pl.ds(offs[b], GCH)]], rows[b], semg)
              for b in range(3)]
        for d in d1:
            d.wait()
        d2 = [pltpu.async_copy(t2_hbm.at[sslab.at[pl.ds(offs[b], GCH)]], rows[b],
                               semg, add=True) for b in range(3)]
        for d in d2:
            d.wait()
        for b in range(3):
            wb = pl.multiple_of(base + offs[b], GCH)
            pltpu.async_copy(rows[b], u_hbm.at[pl.ds(wb, GCH)], semw)
        return carry

    lax.fori_loop(0, 65, grp, 0)
    for b in range(3):
        pltpu.make_async_copy(rows[b], u_hbm.at[pl.ds(0, GCH)], semw).wait()

    @pl.when(nch == 196)
    def _():
        off = pl.multiple_of(195 * GCH, GCH)
        pltpu.sync_copy(t1_hbm.at[dslab.at[pl.ds(off, GCH)]], rows0)
        pltpu.sync_copy(t2_hbm.at[sslab.at[pl.ds(off, GCH)]], rows0, add=True)
        pltpu.sync_copy(rows0, u_hbm.at[pl.ds(pl.multiple_of(base + off, GCH), GCH)])


def _k1(t1, t2, dst, src):
    mesh = plsc.VectorSubcoreMesh(core_axis_name="c", subcore_axis_name="s")
    fn = functools.partial(
        pl.kernel,
        out_type=jax.ShapeDtypeStruct((E, 2 * H), jnp.float32),
        mesh=mesh,
        scratch_types=[
            pltpu.VMEM((GSLAB,), jnp.int32),
            pltpu.VMEM((GSLAB,), jnp.int32),
            pltpu.VMEM((GCH, 2 * H), jnp.float32),
            pltpu.VMEM((GCH, 2 * H), jnp.float32),
            pltpu.VMEM((GCH, 2 * H), jnp.float32),
            pltpu.SemaphoreType.DMA,
            pltpu.SemaphoreType.DMA,
        ],
    )(_gather_body)
    return fn(t1, t2, dst, src)


def _k2_body(u_ref, eat_ref, we_ref, e2w_ref, e2b_ref, ms_ref):
    u = u_ref[...]
    c = lax.dot_general(eat_ref[...], we_ref[...], (((0,), (0,)), ((), ())),
                        preferred_element_type=jnp.float32,
                        precision=lax.Precision.HIGHEST)
    m = _softplus(u[:, 0:H] + c)
    m2 = _softplus(
        jnp.dot(m, e2w_ref[...], preferred_element_type=jnp.float32) + e2b_ref[...])
    parf = u[:, H:H + 1]
    ms_ref[...] = jnp.concatenate([m2 * (1.0 - parf), m2 * parf], axis=1)


def _k2(u, eat, we, e2w, e2b):
    f32 = jnp.float32
    return pl.pallas_call(
        _k2_body,
        grid=(E_STEPS,),
        in_specs=[
            pl.BlockSpec((EB, 2 * H), lambda i: (i, 0)),
            pl.BlockSpec((ED, EB), lambda i: (0, i)),
            pl.BlockSpec((ED, H), lambda i: (0, 0)),
            pl.BlockSpec((H, H), lambda i: (0, 0)),
            pl.BlockSpec((1, H), lambda i: (0, 0)),
        ],
        out_specs=pl.BlockSpec((EB, 2 * H), lambda i: (i, 0)),
        out_shape=jax.ShapeDtypeStruct((E, 2 * H), f32),
    )(u, eat, we, e2w, e2b)


def _scatter_body(ms_hbm, dst_hbm, out0_hbm, out1_hbm,
                  aggr_sh, idxb, lidx, datab):
    c = lax.axis_index("c")
    s = lax.axis_index("s")

    def zr(r, carry):
        for g in range(8):
            datab[r, pl.ds(g * 16, 16)] = jnp.zeros((16,), jnp.float32)
        return carry
    lax.fori_loop(0, GCH, zr, 0)

    zbase = pl.multiple_of(s * ZR, 8)

    def zc(k, carry):
        pltpu.sync_copy(datab, aggr_sh.at[pl.ds(pl.multiple_of(zbase + k * GCH, 8), GCH)])
        return carry
    lax.fori_loop(0, ZR // GCH, zc, 0)
    zrem = ZR - (ZR // GCH) * GCH
    pltpu.sync_copy(datab.at[pl.ds(0, zrem)],
                    aggr_sh.at[pl.ds(pl.multiple_of(zbase + (ZR // GCH) * GCH, 8), zrem)])
    plsc.subcore_barrier()

    cb = s * NCHUNK // 16
    ce = (s + 1) * NCHUNK // 16
    lo = c * PAIRS

    def remap():
        for g in range(8):
            v = idxb[pl.ds(g * 16, 16)]
            local = lax.shift_right_logical(v, 1) - lo
            ok = (local >= 0) & (local < PAIRS)
            trash = PAIRS + (v & 7)
            lidx[pl.ds(g * 16, 16)] = jnp.where(ok, local, trash)

    def chunk(j, carry):
        eoff = pl.multiple_of(j * GCH, GCH)
        pltpu.sync_copy(dst_hbm.at[pl.ds(eoff, GCH)], idxb)
        remap()
        pltpu.sync_copy(ms_hbm.at[pl.ds(eoff, GCH)], datab)
        pltpu.sync_copy(datab, aggr_sh.at[lidx], add=True)
        return carry

    lax.fori_loop(cb, ce, chunk, 0)
    plsc.subcore_barrier()

    obase = pl.multiple_of(s * ZR, 8)

    @pl.when(c == 0)
    def _():
        pltpu.sync_copy(aggr_sh.at[pl.ds(obase, ZR)], out0_hbm.at[pl.ds(obase, ZR)])

    @pl.when(c == 1)
    def _():
        pltpu.sync_copy(aggr_sh.at[pl.ds(obase, ZR)], out1_hbm.at[pl.ds(obase, ZR)])


def _k3(ms, dst):
    f32 = jnp.float32
    mesh = plsc.VectorSubcoreMesh(core_axis_name="c", subcore_axis_name="s")
    fn = functools.partial(
        pl.kernel,
        out_type=[
            jax.ShapeDtypeStruct((PAD_PAIRS, 2 * H), f32),
            jax.ShapeDtypeStruct((PAD_PAIRS, 2 * H), f32),
        ],
        mesh=mesh,
        scratch_types=[
            pltpu.VMEM_SHARED((PAD_PAIRS, 2 * H), f32),
            pltpu.VMEM((GCH,), jnp.int32),
            pltpu.VMEM((GCH,), jnp.int32),
            pltpu.VMEM((GCH, 2 * H), f32),
        ],
    )(_scatter_body)
    return fn(ms, dst)


def _k4_body(h_ref, ab_ref, hh_ref, ag_ref, wa_ref, n2w_ref, n2b_ref,
             hn_ref, st_ref):
    i = pl.program_id(0)
    alpha = ab_ref[0:1, :]
    beta = ab_ref[1:2, :]
    h_eff = h_ref[...] * alpha + beta
    t = _softplus(hh_ref[...]
                  + jnp.dot(ag_ref[...], wa_ref[...], preferred_element_type=jnp.float32, precision=lax.Precision.HIGHEST))
    u = jnp.dot(t, n2w_ref[...], preferred_element_type=jnp.float32, precision=lax.Precision.HIGHEST) + n2b_ref[...]
    hn = u + h_eff
    hn_ref[...] = hn
    upd = jnp.concatenate([
        jnp.sum(hn, axis=0, keepdims=True),
        jnp.sum(hn * hn, axis=0, keepdims=True),
        jnp.zeros((6, H), jnp.float32),
    ], axis=0)

    @pl.when(i == 0)
    def _():
        st_ref[...] = upd

    @pl.when(i > 0)
    def _():
        st_ref[...] += upd


def _k4(h, ab, hh, aggr, wa, n2w, n2b):
    f32 = jnp.float32
    return pl.pallas_call(
        _k4_body,
        grid=(N_STEPS,),
        in_specs=[
            pl.BlockSpec((NB, D), lambda i: (i, 0)),
            pl.BlockSpec((8, H), lambda i: (0, 0)),
            pl.BlockSpec((NB, H), lambda i: (i, 0)),
            pl.BlockSpec((NB, H), lambda i: (i, 0)),
            pl.BlockSpec((H, H), lambda i: (0, 0)),
            pl.BlockSpec((H, D), lambda i: (0, 0)),
            pl.BlockSpec((1, D), lambda i: (0, 0)),
        ],
        out_specs=[
            pl.BlockSpec((NB, D), lambda i: (i, 0)),
            pl.BlockSpec((8, D), lambda i: (0, 0)),
        ],
        out_shape=[
            jax.ShapeDtypeStruct((N, D), f32),
            jax.ShapeDtypeStruct((8, D), f32),
        ],
    )(h, ab, hh, aggr, wa, n2w, n2b)


def _k6_body(h_ref, st_ref, bng_ref, bnb_ref, b_ref, ow1_ref, ob1_ref,
             ow2_ref, ob2_ref, out_ref, sums, counts):
    i = pl.program_id(0)
    s = st_ref[0:1, :]
    sq = st_ref[1:2, :]
    mean = s * (1.0 / N)
    var = sq * (1.0 / N) - mean * mean
    alpha = bng_ref[...] * lax.rsqrt(var + 1e-5)
    beta = bnb_ref[...] - mean * alpha
    h_eff = h_ref[...] * alpha + beta
    bb = b_ref[...]
    oh = (bb == lax.broadcasted_iota(jnp.int32, (NB, G), 1)).astype(jnp.float32)
    dnums = (((0,), (0,)), ((), ()))
    sum_c = lax.dot_general(oh, h_eff, dnums, preferred_element_type=jnp.float32, precision=lax.Precision.HIGHEST)
    cnt_c = lax.dot_general(oh, jnp.ones((NB, 1), jnp.float32), dnums,
                            preferred_element_type=jnp.float32, precision=lax.Precision.HIGHEST)

    @pl.when(i == 0)
    def _():
        sums[...] = sum_c
        counts[...] = cnt_c

    @pl.when(i > 0)
    def _():
        sums[...] += sum_c
        counts[...] += cnt_c

    @pl.when(i == N_STEPS - 1)
    def _():
        pooled = sums[...] / jnp.maximum(counts[...], 1.0)
        o1 = _softplus(jnp.dot(pooled, ow1_ref[...], preferred_element_type=jnp.float32, precision=lax.Precision.HIGHEST)
                       + ob1_ref[...])
        out_ref[...] = jnp.dot(o1, ow2_ref[...], preferred_element_type=jnp.float32, precision=lax.Precision.HIGHEST) + ob2_ref[...]


def _k6(h, stats, bng, bnb, batch2, ow1, ob1, ow2, ob2):
    f32 = jnp.float32
    return pl.pallas_call(
        _k6_body,
        grid=(N_STEPS,),
        in_specs=[
            pl.BlockSpec((NB, D), lambda i: (i, 0)),
            pl.BlockSpec((8, D), lambda i: (0, 0)),
            pl.BlockSpec((1, D), lambda i: (0, 0)),
            pl.BlockSpec((1, D), lambda i: (0, 0)),
            pl.BlockSpec((NB, 1), lambda i: (i, 0)),
            pl.BlockSpec((D, H), lambda i: (0, 0)),
            pl.BlockSpec((1, H), lambda i: (0, 0)),
            pl.BlockSpec((H, 1), lambda i: (0, 0)),
            pl.BlockSpec((1, 1), lambda i: (0, 0)),
        ],
        out_specs=pl.BlockSpec((G, 1), lambda i: (0, 0)),
        out_shape=jax.ShapeDtypeStruct((G, 1), f32),
        scratch_shapes=[
            pltpu.VMEM((G, D), f32),
            pltpu.VMEM((G, 1), f32),
        ],
    )(h, stats, bng, bnb, batch2, ow1, ob1, ow2, ob2)


def kernel(x, edge_index, edge_attr, batch, emb,
           e1w0, e1b0, e2w0, e2b0, n1w0, n1b0, n2w0, n2b0, bng0, bnb0,
           e1w1, e1b1, e2w1, e2b1, n1w1, n1b1, n2w1, n2b1, bng1, bnb1,
           e1w2, e1b2, e2w2, e2b2, n1w2, n1b2, n2w2, n2b2, bng2, bnb2,
           ow1, ob1, ow2, ob2):
    f32 = jnp.float32
    x2 = x.astype(jnp.int32).reshape(N, 1)
    batch2 = batch.astype(jnp.int32).reshape(N, 1)
    src = edge_index[0].astype(jnp.int32)
    dst = edge_index[1].astype(jnp.int32)
    eat = edge_attr.T

    e1w = [e1w0, e1w1, e1w2]
    e1b = [e1b0.reshape(1, H), e1b1.reshape(1, H), e1b2.reshape(1, H)]
    e2w = [e2w0, e2w1, e2w2]
    e2b = [e2b0.reshape(1, H), e2b1.reshape(1, H), e2b2.reshape(1, H)]
    n1w = [n1w0, n1w1, n1w2]
    n1b = [n1b0.reshape(1, H), n1b1.reshape(1, H), n1b2.reshape(1, H)]
    n2w = [n2w0, n2w1, n2w2]
    n2b = [n2b0.reshape(1, D), n2b1.reshape(1, D), n2b2.reshape(1, D)]
    bng = [bng0.reshape(1, D), bng1.reshape(1, D), bng2.reshape(1, D)]
    bnb = [bnb0.reshape(1, D), bnb1.reshape(1, D), bnb2.reshape(1, D)]

    stats = None
    h = None
    for l in range(3):
        wi = e1w[l][0:D]
        wj = e1w[l][D:2 * D]
        we = e1w[l][2 * D:2 * D + ED]
        wh = n1w[l][0:D]
        wa = n1w[l][D:2 * D]
        if l == 0:
            h, t1, t2, hh = _k0(x2, emb, wi, wj, wh, e1b[l], n1b[l])
            ab = jnp.concatenate([jnp.ones((1, H), f32), jnp.zeros((7, H), f32)], axis=0)
        else:
            t1, t2, hh, ab = _k5(h, stats, bng[l - 1], bnb[l - 1], wi, wj, wh,
                                 e1b[l], n1b[l])
        u = _k1(t1, t2, dst, src)
        ms = _k2(u, eat, we, e2w[l], e2b[l])
        p0, p1 = _k3(ms, dst)
        aggr = jnp.concatenate([
            p0.reshape(PAD_PAIRS * 2, H)[0:N // 2],
            p1.reshape(PAD_PAIRS * 2, H)[0:N // 2],
        ], axis=0)
        h, stats = _k4(h, ab, hh, aggr, wa, n2w[l], n2b[l])

    return _k6(h, stats, bng[2], bnb[2], batch2, ow1, ob1.reshape(1, H),
               ow2, ob2.reshape(1, 1))


# speedup vs baseline: 2.4936x; 1.0791x over previous
import functools

import jax
import jax.numpy as jnp
from jax import lax
from jax.experimental import pallas as pl
from jax.experimental.pallas import tpu as pltpu
from jax.experimental.pallas import tpu_sc as plsc

N = 50000
E = 800000
D = 64
ED = 16
H = 64
G = 256

NB = 1000
N_STEPS = N // NB
EB = 6400
E_STEPS = E // EB

NW = 32
GCH = 128
NCHUNK = E // GCH
GSLAB = (NCHUNK // NW + 1) * GCH

PAIRS = N // 4
PAD_PAIRS = 12544
ZR = PAD_PAIRS // 16
SCH = 64
NSCH = E // SCH


def _softplus(x):
    return jnp.maximum(x, 0.0) + jnp.log(1.0 + jnp.exp(-jnp.abs(x)))


def _pq(h_eff, wi_ref, wj_ref, e1b_ref):
    a = jnp.dot(h_eff, wi_ref[...], preferred_element_type=jnp.float32, precision=lax.Precision.HIGHEST) + e1b_ref[...]
    b = jnp.dot(h_eff, wj_ref[...], preferred_element_type=jnp.float32, precision=lax.Precision.HIGHEST)
    par = (lax.broadcasted_iota(jnp.int32, (NB, H), 0) & 1).astype(jnp.float32)
    t1 = jnp.concatenate([a, par], axis=1)
    t2 = jnp.concatenate([b, jnp.zeros((NB, H), jnp.float32)], axis=1)
    return t1, t2


def _k0_body(x_ref, emb_ref, wi_ref, wj_ref, wh_ref, e1b_ref, n1b_ref,
             h_ref, t1_ref, t2_ref, hh_ref):
    xb = x_ref[...]
    oh = (xb == lax.broadcasted_iota(jnp.int32, (NB, 100), 1)).astype(jnp.float32)
    h = jnp.dot(oh, emb_ref[...], preferred_element_type=jnp.float32, precision=lax.Precision.HIGHEST)
    h_ref[...] = h
    t1_ref[...], t2_ref[...] = _pq(h, wi_ref, wj_ref, e1b_ref)
    hh_ref[...] = jnp.dot(h, wh_ref[...], preferred_element_type=jnp.float32, precision=lax.Precision.HIGHEST) + n1b_ref[...]


def _k0(x2, emb, wi, wj, wh, e1b, n1b):
    f32 = jnp.float32
    return pl.pallas_call(
        _k0_body,
        grid=(N_STEPS,),
        in_specs=[
            pl.BlockSpec((NB, 1), lambda i: (i, 0)),
            pl.BlockSpec((100, D), lambda i: (0, 0)),
            pl.BlockSpec((D, H), lambda i: (0, 0)),
            pl.BlockSpec((D, H), lambda i: (0, 0)),
            pl.BlockSpec((D, H), lambda i: (0, 0)),
            pl.BlockSpec((1, H), lambda i: (0, 0)),
            pl.BlockSpec((1, H), lambda i: (0, 0)),
        ],
        out_specs=[
            pl.BlockSpec((NB, D), lambda i: (i, 0)),
            pl.BlockSpec((NB, 2 * H), lambda i: (i, 0)),
            pl.BlockSpec((NB, 2 * H), lambda i: (i, 0)),
            pl.BlockSpec((NB, H), lambda i: (i, 0)),
        ],
        out_shape=[
            jax.ShapeDtypeStruct((N, D), f32),
            jax.ShapeDtypeStruct((N, 2 * H), f32),
            jax.ShapeDtypeStruct((N, 2 * H), f32),
            jax.ShapeDtypeStruct((N, H), f32),
        ],
    )(x2, emb, wi, wj, wh, e1b, n1b)


def _k5_body(h_ref, st_ref, bng_ref, bnb_ref, wi_ref, wj_ref, wh_ref,
             e1b_ref, n1b_ref, t1_ref, t2_ref, hh_ref, ab_ref):
    s = st_ref[0:1, :]
    sq = st_ref[1:2, :]
    mean = s * (1.0 / N)
    var = sq * (1.0 / N) - mean * mean
    alpha = bng_ref[...] * lax.rsqrt(var + 1e-5)
    beta = bnb_ref[...] - mean * alpha
    h_eff = h_ref[...] * alpha + beta
    t1_ref[...], t2_ref[...] = _pq(h_eff, wi_ref, wj_ref, e1b_ref)
    hh_ref[...] = jnp.dot(h_eff, wh_ref[...], preferred_element_type=jnp.float32, precision=lax.Precision.HIGHEST) + n1b_ref[...]
    ab_ref[...] = jnp.concatenate([alpha, beta, jnp.zeros((6, H), jnp.float32)], axis=0)


def _k5(h, stats, bng, bnb, wi, wj, wh, e1b, n1b):
    f32 = jnp.float32
    return pl.pallas_call(
        _k5_body,
        grid=(N_STEPS,),
        in_specs=[
            pl.BlockSpec((NB, D), lambda i: (i, 0)),
            pl.BlockSpec((8, H), lambda i: (0, 0)),
            pl.BlockSpec((1, H), lambda i: (0, 0)),
            pl.BlockSpec((1, H), lambda i: (0, 0)),
            pl.BlockSpec((D, H), lambda i: (0, 0)),
            pl.BlockSpec((D, H), lambda i: (0, 0)),
            pl.BlockSpec((D, H), lambda i: (0, 0)),
            pl.BlockSpec((1, H), lambda i: (0, 0)),
            pl.BlockSpec((1, H), lambda i: (0, 0)),
        ],
        out_specs=[
            pl.BlockSpec((NB, 2 * H), lambda i: (i, 0)),
            pl.BlockSpec((NB, 2 * H), lambda i: (i, 0)),
            pl.BlockSpec((NB, H), lambda i: (i, 0)),
            pl.BlockSpec((8, H), lambda i: (0, 0)),
        ],
        out_shape=[
            jax.ShapeDtypeStruct((N, 2 * H), f32),
            jax.ShapeDtypeStruct((N, 2 * H), f32),
            jax.ShapeDtypeStruct((N, H), f32),
            jax.ShapeDtypeStruct((8, H), f32),
        ],
    )(h, stats, bng, bnb, wi, wj, wh, e1b, n1b)


def _gather_body(t1_hbm, t2_hbm, dst_hbm, src_hbm, u_hbm, dslab, sslab,
                 rows0, rows1, rows2, semg, semw):
    c = lax.axis_index("c")
    s = lax.axis_index("s")
    wid = s * 2 + c
    cb = wid * NCHUNK // NW
    ce = (wid + 1) * NCHUNK // NW
    nch = ce - cb
    base = pl.multiple_of(cb * GCH, GCH)
    pltpu.sync_copy(dst_hbm.at[pl.ds(base, GSLAB - GCH)], dslab.at[pl.ds(0, GSLAB - GCH)])
    pltpu.sync_copy(src_hbm.at[pl.ds(base, GSLAB - GCH)], sslab.at[pl.ds(0, GSLAB - GCH)])

    @pl.when(nch * GCH == GSLAB)
    def _():
        off = pl.multiple_of(base + GSLAB - GCH, GCH)
        pltpu.sync_copy(dst_hbm.at[pl.ds(off, GCH)], dslab.at[pl.ds(GSLAB - GCH, GCH)])
        pltpu.sync_copy(src_hbm.at[pl.ds(off, GCH)], sslab.at[pl.ds(GSLAB - GCH, GCH)])

    rows = (rows0, rows1, rows2)

    def grp(g, carry):
        @pl.when(g > 0)
        def _():
            for b in range(3):
                pltpu.make_async_copy(rows[b], u_hbm.at[pl.ds(0, GCH)], semw).wait()

        offs = [pl.multiple_of((g * 3 + b) * GCH, GCH) for b in range(3)]
        d1 = [pltpu.async_copy(t1_hbm.at[dslab.at[pl.ds(offs[b], GCH)]], rows[b], semg)
              for b in range(3)]
        for d in d1:
            d.wait()
        d2 = [pltpu.async_copy(t2_hbm.at[sslab.at[pl.ds(offs[b], GCH)]], rows[b],
                               semg, add=True) for b in range(3)]
        for d in d2:
            d.wait()
        for b in range(3):
            wb = pl.multiple_of(base + offs[b], GCH)
            pltpu.async_copy(rows[b], u_hbm.at[pl.ds(wb, GCH)], semw)
        return carry

    lax.fori_loop(0, 65, grp, 0)
    for b in range(3):
        pltpu.make_async_copy(rows[b], u_hbm.at[pl.ds(0, GCH)], semw).wait()

    @pl.when(nch == 196)
    def _():
        off = pl.multiple_of(195 * GCH, GCH)
        pltpu.sync_copy(t1_hbm.at[dslab.at[pl.ds(off, GCH)]], rows0)
        pltpu.sync_copy(t2_hbm.at[sslab.at[pl.ds(off, GCH)]], rows0, add=True)
        pltpu.sync_copy(rows0, u_hbm.at[pl.ds(pl.multiple_of(base + off, GCH), GCH)])


def _k1(t1, t2, dst, src):
    mesh = plsc.VectorSubcoreMesh(core_axis_name="c", subcore_axis_name="s")
    fn = functools.partial(
        pl.kernel,
        out_type=jax.ShapeDtypeStruct((E, 2 * H), jnp.float32),
        mesh=mesh,
        scratch_types=[
            pltpu.VMEM((GSLAB,), jnp.int32),
            pltpu.VMEM((GSLAB,), jnp.int32),
            pltpu.VMEM((GCH, 2 * H), jnp.float32),
            pltpu.VMEM((GCH, 2 * H), jnp.float32),
            pltpu.VMEM((GCH, 2 * H), jnp.float32),
            pltpu.SemaphoreType.DMA,
            pltpu.SemaphoreType.DMA,
        ],
    )(_gather_body)
    return fn(t1, t2, dst, src)


def _k2_body(u_ref, eat_ref, we_ref, e2w_ref, e2b_ref, ms_ref):
    u = u_ref[...]
    c = lax.dot_general(eat_ref[...], we_ref[...], (((0,), (0,)), ((), ())),
                        preferred_element_type=jnp.float32,
                        precision=lax.Precision.HIGHEST)
    m = _softplus(u[:, 0:H] + c)
    m2 = _softplus(
        jnp.dot(m, e2w_ref[...], preferred_element_type=jnp.float32) + e2b_ref[...])
    parf = u[:, H:H + 1]
    ms_ref[...] = jnp.concatenate([m2 * (1.0 - parf), m2 * parf], axis=1)


def _k2(u, eat, we, e2w, e2b):
    f32 = jnp.float32
    return pl.pallas_call(
        _k2_body,
        grid=(E_STEPS,),
        in_specs=[
            pl.BlockSpec((EB, 2 * H), lambda i: (i, 0)),
            pl.BlockSpec((ED, EB), lambda i: (0, i)),
            pl.BlockSpec((ED, H), lambda i: (0, 0)),
            pl.BlockSpec((H, H), lambda i: (0, 0)),
            pl.BlockSpec((1, H), lambda i: (0, 0)),
        ],
        out_specs=pl.BlockSpec((EB, 2 * H), lambda i: (i, 0)),
        out_shape=jax.ShapeDtypeStruct((E, 2 * H), f32),
    )(u, eat, we, e2w, e2b)


def _scatter_body(ms_hbm, dst_hbm, out0_hbm, out1_hbm,
                  aggr_sh, idxb, idxb2, lidx, lidx2, datab, datab2,
                  semi, semd, semsc):
    c = lax.axis_index("c")
    s = lax.axis_index("s")

    def zr(r, carry):
        for g in range(8):
            datab[r, pl.ds(g * 16, 16)] = jnp.zeros((16,), jnp.float32)
        return carry
    lax.fori_loop(0, SCH, zr, 0)

    zbase = pl.multiple_of(s * ZR, 8)

    def zc(k, carry):
        pltpu.sync_copy(datab, aggr_sh.at[pl.ds(pl.multiple_of(zbase + k * SCH, 8), SCH)])
        return carry
    lax.fori_loop(0, ZR // SCH, zc, 0)
    zrem = ZR - (ZR // SCH) * SCH
    pltpu.sync_copy(datab.at[pl.ds(0, zrem)],
                    aggr_sh.at[pl.ds(pl.multiple_of(zbase + (ZR // SCH) * SCH, 8), zrem)])
    plsc.subcore_barrier()

    cb = s * NSCH // 16
    ce = (s + 1) * NSCH // 16
    lo = c * PAIRS

    idxs = (idxb, idxb2)
    lidxs = (lidx, lidx2)
    datas = (datab, datab2)

    def remap(b):
        for g in range(SCH // 16):
            v = idxs[b][pl.ds(g * 16, 16)]
            local = lax.shift_right_logical(v, 1) - lo
            ok = (local >= 0) & (local < PAIRS)
            trash = PAIRS + (v & 7)
            lidxs[b][pl.ds(g * 16, 16)] = jnp.where(ok, local, trash)

    def pair(p, carry):
        j0 = cb + 2 * p

        @pl.when(p > 0)
        def _():
            for b in range(2):
                pltpu.make_async_copy(datas[b], aggr_sh.at[pl.ds(0, SCH)], semsc).wait()

        for b in range(2):
            eoff = pl.multiple_of((j0 + b) * SCH, 8)
            pltpu.async_copy(dst_hbm.at[pl.ds(eoff, SCH)], idxs[b], semi)
            pltpu.async_copy(ms_hbm.at[pl.ds(eoff, SCH)], datas[b], semd)
        for b in range(2):
            pltpu.make_async_copy(dst_hbm.at[pl.ds(0, SCH)], idxs[b], semi).wait()
        for b in range(2):
            remap(b)
        for b in range(2):
            pltpu.make_async_copy(ms_hbm.at[pl.ds(0, SCH)], datas[b], semd).wait()
        for b in range(2):
            pltpu.async_copy(datas[b], aggr_sh.at[lidxs[b]], semsc, add=True)
        return carry

    npair = (ce - cb) // 2
    lax.fori_loop(0, npair, pair, 0)
    for b in range(2):
        pltpu.make_async_copy(datas[b], aggr_sh.at[pl.ds(0, SCH)], semsc).wait()

    @pl.when((ce - cb) % 2 == 1)
    def _():
        eoff = pl.multiple_of((ce - 1) * SCH, 8)
        pltpu.sync_copy(dst_hbm.at[pl.ds(eoff, SCH)], idxb)
        remap(0)
        pltpu.sync_copy(ms_hbm.at[pl.ds(eoff, SCH)], datab)
        pltpu.sync_copy(datab, aggr_sh.at[lidx], add=True)

    plsc.subcore_barrier()

    obase = pl.multiple_of(s * ZR, 8)

    @pl.when(c == 0)
    def _():
        pltpu.sync_copy(aggr_sh.at[pl.ds(obase, ZR)], out0_hbm.at[pl.ds(obase, ZR)])

    @pl.when(c == 1)
    def _():
        pltpu.sync_copy(aggr_sh.at[pl.ds(obase, ZR)], out1_hbm.at[pl.ds(obase, ZR)])


def _k3(ms, dst):
    f32 = jnp.float32
    mesh = plsc.VectorSubcoreMesh(core_axis_name="c", subcore_axis_name="s")
    fn = functools.partial(
        pl.kernel,
        out_type=[
            jax.ShapeDtypeStruct((PAD_PAIRS, 2 * H), f32),
            jax.ShapeDtypeStruct((PAD_PAIRS, 2 * H), f32),
        ],
        mesh=mesh,
        scratch_types=[
            pltpu.VMEM_SHARED((PAD_PAIRS, 2 * H), f32),
            pltpu.VMEM((SCH,), jnp.int32),
            pltpu.VMEM((SCH,), jnp.int32),
            pltpu.VMEM((SCH,), jnp.int32),
            pltpu.VMEM((SCH,), jnp.int32),
            pltpu.VMEM((SCH, 2 * H), f32),
            pltpu.VMEM((SCH, 2 * H), f32),
            pltpu.SemaphoreType.DMA,
            pltpu.SemaphoreType.DMA,
            pltpu.SemaphoreType.DMA,
        ],
    )(_scatter_body)
    return fn(ms, dst)


def _k4_body(h_ref, ab_ref, hh_ref, ag_ref, wa_ref, n2w_ref, n2b_ref,
             hn_ref, st_ref):
    i = pl.program_id(0)
    alpha = ab_ref[0:1, :]
    beta = ab_ref[1:2, :]
    h_eff = h_ref[...] * alpha + beta
    t = _softplus(hh_ref[...]
                  + jnp.dot(ag_ref[...], wa_ref[...], preferred_element_type=jnp.float32, precision=lax.Precision.HIGHEST))
    u = jnp.dot(t, n2w_ref[...], preferred_element_type=jnp.float32, precision=lax.Precision.HIGHEST) + n2b_ref[...]
    hn = u + h_eff
    hn_ref[...] = hn
    upd = jnp.concatenate([
        jnp.sum(hn, axis=0, keepdims=True),
        jnp.sum(hn * hn, axis=0, keepdims=True),
        jnp.zeros((6, H), jnp.float32),
    ], axis=0)

    @pl.when(i == 0)
    def _():
        st_ref[...] = upd

    @pl.when(i > 0)
    def _():
        st_ref[...] += upd


def _k4(h, ab, hh, aggr, wa, n2w, n2b):
    f32 = jnp.float32
    return pl.pallas_call(
        _k4_body,
        grid=(N_STEPS,),
        in_specs=[
            pl.BlockSpec((NB, D), lambda i: (i, 0)),
            pl.BlockSpec((8, H), lambda i: (0, 0)),
            pl.BlockSpec((NB, H), lambda i: (i, 0)),
            pl.BlockSpec((NB, H), lambda i: (i, 0)),
            pl.BlockSpec((H, H), lambda i: (0, 0)),
            pl.BlockSpec((H, D), lambda i: (0, 0)),
            pl.BlockSpec((1, D), lambda i: (0, 0)),
        ],
        out_specs=[
            pl.BlockSpec((NB, D), lambda i: (i, 0)),
            pl.BlockSpec((8, D), lambda i: (0, 0)),
        ],
        out_shape=[
            jax.ShapeDtypeStruct((N, D), f32),
            jax.ShapeDtypeStruct((8, D), f32),
        ],
    )(h, ab, hh, aggr, wa, n2w, n2b)


def _k6_body(h_ref, st_ref, bng_ref, bnb_ref, b_ref, ow1_ref, ob1_ref,
             ow2_ref, ob2_ref, out_ref, sums, counts):
    i = pl.program_id(0)
    s = st_ref[0:1, :]
    sq = st_ref[1:2, :]
    mean = s * (1.0 / N)
    var = sq * (1.0 / N) - mean * mean
    alpha = bng_ref[...] * lax.rsqrt(var + 1e-5)
    beta = bnb_ref[...] - mean * alpha
    h_eff = h_ref[...] * alpha + beta
    bb = b_ref[...]
    oh = (bb == lax.broadcasted_iota(jnp.int32, (NB, G), 1)).astype(jnp.float32)
    dnums = (((0,), (0,)), ((), ()))
    sum_c = lax.dot_general(oh, h_eff, dnums, preferred_element_type=jnp.float32, precision=lax.Precision.HIGHEST)
    cnt_c = lax.dot_general(oh, jnp.ones((NB, 1), jnp.float32), dnums,
                            preferred_element_type=jnp.float32, precision=lax.Precision.HIGHEST)

    @pl.when(i == 0)
    def _():
        sums[...] = sum_c
        counts[...] = cnt_c

    @pl.when(i > 0)
    def _():
        sums[...] += sum_c
        counts[...] += cnt_c

    @pl.when(i == N_STEPS - 1)
    def _():
        pooled = sums[...] / jnp.maximum(counts[...], 1.0)
        o1 = _softplus(jnp.dot(pooled, ow1_ref[...], preferred_element_type=jnp.float32, precision=lax.Precision.HIGHEST)
                       + ob1_ref[...])
        out_ref[...] = jnp.dot(o1, ow2_ref[...], preferred_element_type=jnp.float32, precision=lax.Precision.HIGHEST) + ob2_ref[...]


def _k6(h, stats, bng, bnb, batch2, ow1, ob1, ow2, ob2):
    f32 = jnp.float32
    return pl.pallas_call(
        _k6_body,
        grid=(N_STEPS,),
        in_specs=[
            pl.BlockSpec((NB, D), lambda i: (i, 0)),
            pl.BlockSpec((8, D), lambda i: (0, 0)),
            pl.BlockSpec((1, D), lambda i: (0, 0)),
            pl.BlockSpec((1, D), lambda i: (0, 0)),
            pl.BlockSpec((NB, 1), lambda i: (i, 0)),
            pl.BlockSpec((D, H), lambda i: (0, 0)),
            pl.BlockSpec((1, H), lambda i: (0, 0)),
            pl.BlockSpec((H, 1), lambda i: (0, 0)),
            pl.BlockSpec((1, 1), lambda i: (0, 0)),
        ],
        out_specs=pl.BlockSpec((G, 1), lambda i: (0, 0)),
        out_shape=jax.ShapeDtypeStruct((G, 1), f32),
        scratch_shapes=[
            pltpu.VMEM((G, D), f32),
            pltpu.VMEM((G, 1), f32),
        ],
    )(h, stats, bng, bnb, batch2, ow1, ob1, ow2, ob2)


def kernel(x, edge_index, edge_attr, batch, emb,
           e1w0, e1b0, e2w0, e2b0, n1w0, n1b0, n2w0, n2b0, bng0, bnb0,
           e1w1, e1b1, e2w1, e2b1, n1w1, n1b1, n2w1, n2b1, bng1, bnb1,
           e1w2, e1b2, e2w2, e2b2, n1w2, n1b2, n2w2, n2b2, bng2, bnb2,
           ow1, ob1, ow2, ob2):
    f32 = jnp.float32
    x2 = x.astype(jnp.int32).reshape(N, 1)
    batch2 = batch.astype(jnp.int32).reshape(N, 1)
    src = edge_index[0].astype(jnp.int32)
    dst = edge_index[1].astype(jnp.int32)
    eat = edge_attr.T

    e1w = [e1w0, e1w1, e1w2]
    e1b = [e1b0.reshape(1, H), e1b1.reshape(1, H), e1b2.reshape(1, H)]
    e2w = [e2w0, e2w1, e2w2]
    e2b = [e2b0.reshape(1, H), e2b1.reshape(1, H), e2b2.reshape(1, H)]
    n1w = [n1w0, n1w1, n1w2]
    n1b = [n1b0.reshape(1, H), n1b1.reshape(1, H), n1b2.reshape(1, H)]
    n2w = [n2w0, n2w1, n2w2]
    n2b = [n2b0.reshape(1, D), n2b1.reshape(1, D), n2b2.reshape(1, D)]
    bng = [bng0.reshape(1, D), bng1.reshape(1, D), bng2.reshape(1, D)]
    bnb = [bnb0.reshape(1, D), bnb1.reshape(1, D), bnb2.reshape(1, D)]

    stats = None
    h = None
    for l in range(3):
        wi = e1w[l][0:D]
        wj = e1w[l][D:2 * D]
        we = e1w[l][2 * D:2 * D + ED]
        wh = n1w[l][0:D]
        wa = n1w[l][D:2 * D]
        if l == 0:
            h, t1, t2, hh = _k0(x2, emb, wi, wj, wh, e1b[l], n1b[l])
            ab = jnp.concatenate([jnp.ones((1, H), f32), jnp.zeros((7, H), f32)], axis=0)
        else:
            t1, t2, hh, ab = _k5(h, stats, bng[l - 1], bnb[l - 1], wi, wj, wh,
                                 e1b[l], n1b[l])
        u = _k1(t1, t2, dst, src)
        ms = _k2(u, eat, we, e2w[l], e2b[l])
        p0, p1 = _k3(ms, dst)
        aggr = jnp.concatenate([
            p0.reshape(PAD_PAIRS * 2, H)[0:N // 2],
            p1.reshape(PAD_PAIRS * 2, H)[0:N // 2],
        ], axis=0)
        h, stats = _k4(h, ab, hh, aggr, wa, n2w[l], n2b[l])

    return _k6(h, stats, bng[2], bnb[2], batch2, ow1, ob1.reshape(1, H),
               ow2, ob2.reshape(1, 1))


# speedup vs baseline: 2.7394x; 1.0986x over previous
import functools

import jax
import jax.numpy as jnp
from jax import lax
from jax.experimental import pallas as pl
from jax.experimental.pallas import tpu as pltpu
from jax.experimental.pallas import tpu_sc as plsc

N = 50000
E = 800000
D = 64
ED = 16
H = 64
G = 256

NB = 1000
N_STEPS = N // NB
EB = 6400
E_STEPS = E // EB

NW = 32
GCH = 128
NCHUNK = E // GCH
GSLAB = (NCHUNK // NW + 1) * GCH

PAIRS = N // 4
PAD_PAIRS = 12544
ZR = PAD_PAIRS // 16
SCH = 64
NSCH = E // SCH


def _softplus(x):
    return jnp.maximum(x, 0.0) + jnp.log(1.0 + jnp.exp(-jnp.abs(x)))


def _pq(h_eff, wi_ref, wj_ref, e1b_ref):
    a = jnp.dot(h_eff, wi_ref[...], preferred_element_type=jnp.float32, precision=lax.Precision.HIGHEST) + e1b_ref[...]
    b = jnp.dot(h_eff, wj_ref[...], preferred_element_type=jnp.float32, precision=lax.Precision.HIGHEST)
    par = (lax.broadcasted_iota(jnp.int32, (NB, H), 0) & 1).astype(jnp.float32)
    t1 = jnp.concatenate([a, par], axis=1)
    t2 = jnp.concatenate([b, jnp.zeros((NB, H), jnp.float32)], axis=1)
    return t1, t2


def _k0_body(x_ref, emb_ref, wi_ref, wj_ref, wh_ref, e1b_ref, n1b_ref,
             h_ref, t1_ref, t2_ref, hh_ref):
    xb = x_ref[...]
    oh = (xb == lax.broadcasted_iota(jnp.int32, (NB, 100), 1)).astype(jnp.float32)
    h = jnp.dot(oh, emb_ref[...], preferred_element_type=jnp.float32, precision=lax.Precision.HIGHEST)
    h_ref[...] = h
    t1_ref[...], t2_ref[...] = _pq(h, wi_ref, wj_ref, e1b_ref)
    hh_ref[...] = jnp.dot(h, wh_ref[...], preferred_element_type=jnp.float32, precision=lax.Precision.HIGHEST) + n1b_ref[...]


def _k0(x2, emb, wi, wj, wh, e1b, n1b):
    f32 = jnp.float32
    return pl.pallas_call(
        _k0_body,
        grid=(N_STEPS,),
        in_specs=[
            pl.BlockSpec((NB, 1), lambda i: (i, 0)),
            pl.BlockSpec((100, D), lambda i: (0, 0)),
            pl.BlockSpec((D, H), lambda i: (0, 0)),
            pl.BlockSpec((D, H), lambda i: (0, 0)),
            pl.BlockSpec((D, H), lambda i: (0, 0)),
            pl.BlockSpec((1, H), lambda i: (0, 0)),
            pl.BlockSpec((1, H), lambda i: (0, 0)),
        ],
        out_specs=[
            pl.BlockSpec((NB, D), lambda i: (i, 0)),
            pl.BlockSpec((NB, 2 * H), lambda i: (i, 0)),
            pl.BlockSpec((NB, 2 * H), lambda i: (i, 0)),
            pl.BlockSpec((NB, H), lambda i: (i, 0)),
        ],
        out_shape=[
            jax.ShapeDtypeStruct((N, D), f32),
            jax.ShapeDtypeStruct((N, 2 * H), f32),
            jax.ShapeDtypeStruct((N, 2 * H), f32),
            jax.ShapeDtypeStruct((N, H), f32),
        ],
    )(x2, emb, wi, wj, wh, e1b, n1b)


def _k5_body(h_ref, st_ref, bng_ref, bnb_ref, wi_ref, wj_ref, wh_ref,
             e1b_ref, n1b_ref, t1_ref, t2_ref, hh_ref, ab_ref):
    s = st_ref[0:1, :]
    sq = st_ref[1:2, :]
    mean = s * (1.0 / N)
    var = sq * (1.0 / N) - mean * mean
    alpha = bng_ref[...] * lax.rsqrt(var + 1e-5)
    beta = bnb_ref[...] - mean * alpha
    h_eff = h_ref[...] * alpha + beta
    t1_ref[...], t2_ref[...] = _pq(h_eff, wi_ref, wj_ref, e1b_ref)
    hh_ref[...] = jnp.dot(h_eff, wh_ref[...], preferred_element_type=jnp.float32, precision=lax.Precision.HIGHEST) + n1b_ref[...]
    ab_ref[...] = jnp.concatenate([alpha, beta, jnp.zeros((6, H), jnp.float32)], axis=0)


def _k5(h, stats, bng, bnb, wi, wj, wh, e1b, n1b):
    f32 = jnp.float32
    return pl.pallas_call(
        _k5_body,
        grid=(N_STEPS,),
        in_specs=[
            pl.BlockSpec((NB, D), lambda i: (i, 0)),
            pl.BlockSpec((8, H), lambda i: (0, 0)),
            pl.BlockSpec((1, H), lambda i: (0, 0)),
            pl.BlockSpec((1, H), lambda i: (0, 0)),
            pl.BlockSpec((D, H), lambda i: (0, 0)),
            pl.BlockSpec((D, H), lambda i: (0, 0)),
            pl.BlockSpec((D, H), lambda i: (0, 0)),
            pl.BlockSpec((1, H), lambda i: (0, 0)),
            pl.BlockSpec((1, H), lambda i: (0, 0)),
        ],
        out_specs=[
            pl.BlockSpec((NB, 2 * H), lambda i: (i, 0)),
            pl.BlockSpec((NB, 2 * H), lambda i: (i, 0)),
            pl.BlockSpec((NB, H), lambda i: (i, 0)),
            pl.BlockSpec((8, H), lambda i: (0, 0)),
        ],
        out_shape=[
            jax.ShapeDtypeStruct((N, 2 * H), f32),
            jax.ShapeDtypeStruct((N, 2 * H), f32),
            jax.ShapeDtypeStruct((N, H), f32),
            jax.ShapeDtypeStruct((8, H), f32),
        ],
    )(h, stats, bng, bnb, wi, wj, wh, e1b, n1b)


def _gather_body(t1_hbm, t2_hbm, dst_hbm, src_hbm, u_hbm, dslab, sslab,
                 rows0, rows1, rows2, semg, semw):
    c = lax.axis_index("c")
    s = lax.axis_index("s")
    wid = s * 2 + c
    cb = wid * NCHUNK // NW
    ce = (wid + 1) * NCHUNK // NW
    nch = ce - cb
    base = pl.multiple_of(cb * GCH, GCH)
    pltpu.sync_copy(dst_hbm.at[pl.ds(base, GSLAB - GCH)], dslab.at[pl.ds(0, GSLAB - GCH)])
    pltpu.sync_copy(src_hbm.at[pl.ds(base, GSLAB - GCH)], sslab.at[pl.ds(0, GSLAB - GCH)])

    @pl.when(nch * GCH == GSLAB)
    def _():
        off = pl.multiple_of(base + GSLAB - GCH, GCH)
        pltpu.sync_copy(dst_hbm.at[pl.ds(off, GCH)], dslab.at[pl.ds(GSLAB - GCH, GCH)])
        pltpu.sync_copy(src_hbm.at[pl.ds(off, GCH)], sslab.at[pl.ds(GSLAB - GCH, GCH)])

    rows = (rows0, rows1, rows2)

    def grp(g, carry):
        @pl.when(g > 0)
        def _():
            for b in range(3):
                pltpu.make_async_copy(rows[b], u_hbm.at[pl.ds(0, GCH)], semw).wait()

        offs = [pl.multiple_of((g * 3 + b) * GCH, GCH) for b in range(3)]
        d1 = [pltpu.async_copy(t1_hbm.at[dslab.at[pl.ds(offs[b], GCH)]], rows[b], semg)
              for b in range(3)]
        for d in d1:
            d.wait()
        d2 = [pltpu.async_copy(t2_hbm.at[sslab.at[pl.ds(offs[b], GCH)]], rows[b],
                               semg, add=True) for b in range(3)]
        for d in d2:
            d.wait()
        for b in range(3):
            wb = pl.multiple_of(base + offs[b], GCH)
            pltpu.async_copy(rows[b], u_hbm.at[pl.ds(wb, GCH)], semw)
        return carry

    lax.fori_loop(0, 65, grp, 0)
    for b in range(3):
        pltpu.make_async_copy(rows[b], u_hbm.at[pl.ds(0, GCH)], semw).wait()

    @pl.when(nch == 196)
    def _():
        off = pl.multiple_of(195 * GCH, GCH)
        pltpu.sync_copy(t1_hbm.at[dslab.at[pl.ds(off, GCH)]], rows0)
        pltpu.sync_copy(t2_hbm.at[sslab.at[pl.ds(off, GCH)]], rows0, add=True)
        pltpu.sync_copy(rows0, u_hbm.at[pl.ds(pl.multiple_of(base + off, GCH), GCH)])


def _k1(t1, t2, dst, src):
    mesh = plsc.VectorSubcoreMesh(core_axis_name="c", subcore_axis_name="s")
    fn = functools.partial(
        pl.kernel,
        out_type=jax.ShapeDtypeStruct((E, 2 * H), jnp.float32),
        mesh=mesh,
        scratch_types=[
            pltpu.VMEM((GSLAB,), jnp.int32),
            pltpu.VMEM((GSLAB,), jnp.int32),
            pltpu.VMEM((GCH, 2 * H), jnp.float32),
            pltpu.VMEM((GCH, 2 * H), jnp.float32),
            pltpu.VMEM((GCH, 2 * H), jnp.float32),
            pltpu.SemaphoreType.DMA,
            pltpu.SemaphoreType.DMA,
        ],
    )(_gather_body)
    return fn(t1, t2, dst, src)


def _k2_body(u_ref, eat_ref, we_ref, e2w_ref, e2b_ref, ms_ref):
    u = u_ref[...]
    c = lax.dot_general(eat_ref[...], we_ref[...], (((0,), (0,)), ((), ())),
                        preferred_element_type=jnp.float32)
    m = _softplus(u[:, 0:H] + c)
    m2 = _softplus(
        jnp.dot(m, e2w_ref[...], preferred_element_type=jnp.float32) + e2b_ref[...])
    parf = u[:, H:H + 1]
    ms_ref[...] = jnp.concatenate([m2 * (1.0 - parf), m2 * parf], axis=1)


def _k2(u, eat, we, e2w, e2b):
    f32 = jnp.float32
    return pl.pallas_call(
        _k2_body,
        grid=(E_STEPS,),
        in_specs=[
            pl.BlockSpec((EB, 2 * H), lambda i: (i, 0)),
            pl.BlockSpec((ED, EB), lambda i: (0, i)),
            pl.BlockSpec((ED, H), lambda i: (0, 0)),
            pl.BlockSpec((H, H), lambda i: (0, 0)),
            pl.BlockSpec((1, H), lambda i: (0, 0)),
        ],
        out_specs=pl.BlockSpec((EB, 2 * H), lambda i: (i, 0)),
        out_shape=jax.ShapeDtypeStruct((E, 2 * H), f32),
    )(u, eat, we, e2w, e2b)


def _scatter_body(ms_hbm, dst_hbm, out0_hbm, out1_hbm,
                  aggr_sh, idxb, idxb2, idxb3, lidx, lidx2, lidx3,
                  datab, datab2, datab3, semi, semd, semsc):
    c = lax.axis_index("c")
    s = lax.axis_index("s")

    def zr(r, carry):
        for g in range(8):
            datab[r, pl.ds(g * 16, 16)] = jnp.zeros((16,), jnp.float32)
        return carry
    lax.fori_loop(0, SCH, zr, 0)

    zbase = pl.multiple_of(s * ZR, 8)

    def zc(k, carry):
        pltpu.sync_copy(datab, aggr_sh.at[pl.ds(pl.multiple_of(zbase + k * SCH, 8), SCH)])
        return carry
    lax.fori_loop(0, ZR // SCH, zc, 0)
    zrem = ZR - (ZR // SCH) * SCH
    pltpu.sync_copy(datab.at[pl.ds(0, zrem)],
                    aggr_sh.at[pl.ds(pl.multiple_of(zbase + (ZR // SCH) * SCH, 8), zrem)])
    plsc.subcore_barrier()

    cb = s * NSCH // 16
    ce = (s + 1) * NSCH // 16
    lo = c * PAIRS

    idxs = (idxb, idxb2, idxb3)
    lidxs = (lidx, lidx2, lidx3)
    datas = (datab, datab2, datab3)
    NBUF = 3

    def remap(b):
        for g in range(SCH // 16):
            v = idxs[b][pl.ds(g * 16, 16)]
            local = lax.shift_right_logical(v, 1) - lo
            ok = (local >= 0) & (local < PAIRS)
            trash = PAIRS + (v & 7)
            lidxs[b][pl.ds(g * 16, 16)] = jnp.where(ok, local, trash)

    def grp(p, carry):
        j0 = cb + NBUF * p

        @pl.when(p > 0)
        def _():
            for b in range(NBUF):
                pltpu.make_async_copy(datas[b], aggr_sh.at[pl.ds(0, SCH)], semsc).wait()

        for b in range(NBUF):
            eoff = pl.multiple_of((j0 + b) * SCH, 8)
            pltpu.async_copy(dst_hbm.at[pl.ds(eoff, SCH)], idxs[b], semi)
            pltpu.async_copy(ms_hbm.at[pl.ds(eoff, SCH)], datas[b], semd)
        for b in range(NBUF):
            pltpu.make_async_copy(dst_hbm.at[pl.ds(0, SCH)], idxs[b], semi).wait()
        for b in range(NBUF):
            remap(b)
        for b in range(NBUF):
            pltpu.make_async_copy(ms_hbm.at[pl.ds(0, SCH)], datas[b], semd).wait()
        for b in range(NBUF):
            pltpu.async_copy(datas[b], aggr_sh.at[lidxs[b]], semsc, add=True)
        return carry

    ngrp = (ce - cb) // NBUF
    lax.fori_loop(0, ngrp, grp, 0)
    for b in range(NBUF):
        pltpu.make_async_copy(datas[b], aggr_sh.at[pl.ds(0, SCH)], semsc).wait()

    def tail(j, carry):
        eoff = pl.multiple_of(j * SCH, 8)
        pltpu.sync_copy(dst_hbm.at[pl.ds(eoff, SCH)], idxb)
        remap(0)
        pltpu.sync_copy(ms_hbm.at[pl.ds(eoff, SCH)], datab)
        pltpu.sync_copy(datab, aggr_sh.at[lidx], add=True)
        return carry

    lax.fori_loop(cb + ngrp * NBUF, ce, tail, 0)
    plsc.subcore_barrier()

    obase = pl.multiple_of(s * ZR, 8)

    @pl.when(c == 0)
    def _():
        pltpu.sync_copy(aggr_sh.at[pl.ds(obase, ZR)], out0_hbm.at[pl.ds(obase, ZR)])

    @pl.when(c == 1)
    def _():
        pltpu.sync_copy(aggr_sh.at[pl.ds(obase, ZR)], out1_hbm.at[pl.ds(obase, ZR)])


def _k3(ms, dst):
    f32 = jnp.float32
    mesh = plsc.VectorSubcoreMesh(core_axis_name="c", subcore_axis_name="s")
    fn = functools.partial(
        pl.kernel,
        out_type=[
            jax.ShapeDtypeStruct((PAD_PAIRS, 2 * H), f32),
            jax.ShapeDtypeStruct((PAD_PAIRS, 2 * H), f32),
        ],
        mesh=mesh,
        scratch_types=[
            pltpu.VMEM_SHARED((PAD_PAIRS, 2 * H), f32),
            pltpu.VMEM((SCH,), jnp.int32),
            pltpu.VMEM((SCH,), jnp.int32),
            pltpu.VMEM((SCH,), jnp.int32),
            pltpu.VMEM((SCH,), jnp.int32),
            pltpu.VMEM((SCH,), jnp.int32),
            pltpu.VMEM((SCH,), jnp.int32),
            pltpu.VMEM((SCH, 2 * H), f32),
            pltpu.VMEM((SCH, 2 * H), f32),
            pltpu.VMEM((SCH, 2 * H), f32),
            pltpu.SemaphoreType.DMA,
            pltpu.SemaphoreType.DMA,
            pltpu.SemaphoreType.DMA,
        ],
    )(_scatter_body)
    return fn(ms, dst)


def _k4_body(h_ref, ab_ref, hh_ref, ag_ref, wa_ref, n2w_ref, n2b_ref,
             hn_ref, st_ref):
    i = pl.program_id(0)
    alpha = ab_ref[0:1, :]
    beta = ab_ref[1:2, :]
    h_eff = h_ref[...] * alpha + beta
    t = _softplus(hh_ref[...]
                  + jnp.dot(ag_ref[...], wa_ref[...], preferred_element_type=jnp.float32, precision=lax.Precision.HIGHEST))
    u = jnp.dot(t, n2w_ref[...], preferred_element_type=jnp.float32, precision=lax.Precision.HIGHEST) + n2b_ref[...]
    hn = u + h_eff
    hn_ref[...] = hn
    upd = jnp.concatenate([
        jnp.sum(hn, axis=0, keepdims=True),
        jnp.sum(hn * hn, axis=0, keepdims=True),
        jnp.zeros((6, H), jnp.float32),
    ], axis=0)

    @pl.when(i == 0)
    def _():
        st_ref[...] = upd

    @pl.when(i > 0)
    def _():
        st_ref[...] += upd


def _k4(h, ab, hh, aggr, wa, n2w, n2b):
    f32 = jnp.float32
    return pl.pallas_call(
        _k4_body,
        grid=(N_STEPS,),
        in_specs=[
            pl.BlockSpec((NB, D), lambda i: (i, 0)),
            pl.BlockSpec((8, H), lambda i: (0, 0)),
            pl.BlockSpec((NB, H), lambda i: (i, 0)),
            pl.BlockSpec((NB, H), lambda i: (i, 0)),
            pl.BlockSpec((H, H), lambda i: (0, 0)),
            pl.BlockSpec((H, D), lambda i: (0, 0)),
            pl.BlockSpec((1, D), lambda i: (0, 0)),
        ],
        out_specs=[
            pl.BlockSpec((NB, D), lambda i: (i, 0)),
            pl.BlockSpec((8, D), lambda i: (0, 0)),
        ],
        out_shape=[
            jax.ShapeDtypeStruct((N, D), f32),
            jax.ShapeDtypeStruct((8, D), f32),
        ],
    )(h, ab, hh, aggr, wa, n2w, n2b)


def _k6_body(h_ref, st_ref, bng_ref, bnb_ref, b_ref, ow1_ref, ob1_ref,
             ow2_ref, ob2_ref, out_ref, sums, counts):
    i = pl.program_id(0)
    s = st_ref[0:1, :]
    sq = st_ref[1:2, :]
    mean = s * (1.0 / N)
    var = sq * (1.0 / N) - mean * mean
    alpha = bng_ref[...] * lax.rsqrt(var + 1e-5)
    beta = bnb_ref[...] - mean * alpha
    h_eff = h_ref[...] * alpha + beta
    bb = b_ref[...]
    oh = (bb == lax.broadcasted_iota(jnp.int32, (NB, G), 1)).astype(jnp.float32)
    dnums = (((0,), (0,)), ((), ()))
    sum_c = lax.dot_general(oh, h_eff, dnums, preferred_element_type=jnp.float32, precision=lax.Precision.HIGHEST)
    cnt_c = lax.dot_general(oh, jnp.ones((NB, 1), jnp.float32), dnums,
                            preferred_element_type=jnp.float32, precision=lax.Precision.HIGHEST)

    @pl.when(i == 0)
    def _():
        sums[...] = sum_c
        counts[...] = cnt_c

    @pl.when(i > 0)
    def _():
        sums[...] += sum_c
        counts[...] += cnt_c

    @pl.when(i == N_STEPS - 1)
    def _():
        pooled = sums[...] / jnp.maximum(counts[...], 1.0)
        o1 = _softplus(jnp.dot(pooled, ow1_ref[...], preferred_element_type=jnp.float32, precision=lax.Precision.HIGHEST)
                       + ob1_ref[...])
        out_ref[...] = jnp.dot(o1, ow2_ref[...], preferred_element_type=jnp.float32, precision=lax.Precision.HIGHEST) + ob2_ref[...]


def _k6(h, stats, bng, bnb, batch2, ow1, ob1, ow2, ob2):
    f32 = jnp.float32
    return pl.pallas_call(
        _k6_body,
        grid=(N_STEPS,),
        in_specs=[
            pl.BlockSpec((NB, D), lambda i: (i, 0)),
            pl.BlockSpec((8, D), lambda i: (0, 0)),
            pl.BlockSpec((1, D), lambda i: (0, 0)),
            pl.BlockSpec((1, D), lambda i: (0, 0)),
            pl.BlockSpec((NB, 1), lambda i: (i, 0)),
            pl.BlockSpec((D, H), lambda i: (0, 0)),
            pl.BlockSpec((1, H), lambda i: (0, 0)),
            pl.BlockSpec((H, 1), lambda i: (0, 0)),
            pl.BlockSpec((1, 1), lambda i: (0, 0)),
        ],
        out_specs=pl.BlockSpec((G, 1), lambda i: (0, 0)),
        out_shape=jax.ShapeDtypeStruct((G, 1), f32),
        scratch_shapes=[
            pltpu.VMEM((G, D), f32),
            pltpu.VMEM((G, 1), f32),
        ],
    )(h, stats, bng, bnb, batch2, ow1, ob1, ow2, ob2)


def kernel(x, edge_index, edge_attr, batch, emb,
           e1w0, e1b0, e2w0, e2b0, n1w0, n1b0, n2w0, n2b0, bng0, bnb0,
           e1w1, e1b1, e2w1, e2b1, n1w1, n1b1, n2w1, n2b1, bng1, bnb1,
           e1w2, e1b2, e2w2, e2b2, n1w2, n1b2, n2w2, n2b2, bng2, bnb2,
           ow1, ob1, ow2, ob2):
    f32 = jnp.float32
    x2 = x.astype(jnp.int32).reshape(N, 1)
    batch2 = batch.astype(jnp.int32).reshape(N, 1)
    src = edge_index[0].astype(jnp.int32)
    dst = edge_index[1].astype(jnp.int32)
    eat = edge_attr.T

    e1w = [e1w0, e1w1, e1w2]
    e1b = [e1b0.reshape(1, H), e1b1.reshape(1, H), e1b2.reshape(1, H)]
    e2w = [e2w0, e2w1, e2w2]
    e2b = [e2b0.reshape(1, H), e2b1.reshape(1, H), e2b2.reshape(1, H)]
    n1w = [n1w0, n1w1, n1w2]
    n1b = [n1b0.reshape(1, H), n1b1.reshape(1, H), n1b2.reshape(1, H)]
    n2w = [n2w0, n2w1, n2w2]
    n2b = [n2b0.reshape(1, D), n2b1.reshape(1, D), n2b2.reshape(1, D)]
    bng = [bng0.reshape(1, D), bng1.reshape(1, D), bng2.reshape(1, D)]
    bnb = [bnb0.reshape(1, D), bnb1.reshape(1, D), bnb2.reshape(1, D)]

    stats = None
    h = None
    for l in range(3):
        wi = e1w[l][0:D]
        wj = e1w[l][D:2 * D]
        we = e1w[l][2 * D:2 * D + ED]
        wh = n1w[l][0:D]
        wa = n1w[l][D:2 * D]
        if l == 0:
            h, t1, t2, hh = _k0(x2, emb, wi, wj, wh, e1b[l], n1b[l])
            ab = jnp.concatenate([jnp.ones((1, H), f32), jnp.zeros((7, H), f32)], axis=0)
        else:
            t1, t2, hh, ab = _k5(h, stats, bng[l - 1], bnb[l - 1], wi, wj, wh,
                                 e1b[l], n1b[l])
        u = _k1(t1, t2, dst, src)
        ms = _k2(u, eat, we, e2w[l], e2b[l])
        p0, p1 = _k3(ms, dst)
        aggr = jnp.concatenate([
            p0.reshape(PAD_PAIRS * 2, H)[0:N // 2],
            p1.reshape(PAD_PAIRS * 2, H)[0:N // 2],
        ], axis=0)
        h, stats = _k4(h, ab, hh, aggr, wa, n2w[l], n2b[l])

    return _k6(h, stats, bng[2], bnb[2], batch2, ow1, ob1.reshape(1, H),
               ow2, ob2.reshape(1, 1))
